# Initial kernel scaffold; baseline (speedup 1.0000x reference)
import functools

import jax
import jax.numpy as jnp
import numpy as np
from jax import lax
from jax.experimental import pallas as pl
from jax.experimental.pallas import tpu as pltpu
from jax.experimental.pallas import tpu_sc as plsc

_N = 10000
_E = 160000
_TW = 25
_NV = 2
_D = 128
_L = 4

_NP = 10240
_EP = 163840
_NB = 1280
_EB = 2048
_CH = 128
_NC = 2
_NS = 16
_NWORK = _NC * _NS
_CPW = _EP // (_NWORK * _CH)
_RPT = _NP // _NS
_DUMP = _N

_HIGH = lax.Precision.HIGHEST


def _silu(x):
    return x * (1.0 / (1.0 + jnp.exp(-x)))


def _dot(a, b):
    return lax.dot_general(a, b, (((1,), (0,)), ((), ())),
                           precision=_HIGH, preferred_element_type=jnp.float32)


def _enc_body(u_ref, pos_ref, var_ref, w1u, w1p, w1v, b1, w2, b2, x_out):
    h = _dot(u_ref[...], w1u[...]) + _dot(pos_ref[...], w1p[...]) \
        + _dot(var_ref[...], w1v[...]) + b1[...]
    h = _silu(h)
    x_out[...] = _silu(_dot(h, w2[...]) + b2[...])


def _encoder(u, pos, var, w1u, w1p, w1v, b1, w2, b2):
    g = _NP // _NB
    row = lambda c: pl.BlockSpec((_NB, c), lambda i: (i, 0))
    full = lambda r, c: pl.BlockSpec((r, c), lambda i: (0, 0))
    return pl.pallas_call(
        _enc_body,
        grid=(g,),
        in_specs=[row(_TW), row(1), row(_NV), full(_TW, _D), full(1, _D),
                  full(_NV, _D), full(1, _D), full(_D, _D), full(1, _D)],
        out_specs=row(_D),
        out_shape=jax.ShapeDtypeStruct((_NP, _D), jnp.float32),
    )(u, pos, var, w1u, w1p, w1v, b1, w2, b2)


def _pre_body_norm(y_ref, st_ref, u_ref, pos_ref, var_ref,
                   wxi, wxj, wdu, wdp, wvar, bm, xn_out, ad_out, as_out):
    mean = st_ref[0:1, :] * (1.0 / _N)
    ex2 = st_ref[1:2, :] * (1.0 / _N)
    rstd = lax.rsqrt(ex2 - mean * mean + 1e-5)
    xn = (y_ref[...] - mean) * rstd
    t = _dot(u_ref[...], wdu[...]) + _dot(pos_ref[...], wdp[...])
    xn_out[...] = xn
    ad_out[...] = _dot(xn, wxi[...]) + t + _dot(var_ref[...], wvar[...]) + bm[...]
    as_out[...] = _dot(xn, wxj[...]) - t


def _pre_body_raw(y_ref, u_ref, pos_ref, var_ref,
                  wxi, wxj, wdu, wdp, wvar, bm, xn_out, ad_out, as_out):
    xn = y_ref[...]
    t = _dot(u_ref[...], wdu[...]) + _dot(pos_ref[...], wdp[...])
    xn_out[...] = xn
    ad_out[...] = _dot(xn, wxi[...]) + t + _dot(var_ref[...], wvar[...]) + bm[...]
    as_out[...] = _dot(xn, wxj[...]) - t


def _pre(y, stats, u, pos, var, wxi, wxj, wdu, wdp, wvar, bm):
    g = _NP // _NB
    row = lambda c: pl.BlockSpec((_NB, c), lambda i: (i, 0))
    full = lambda r, c: pl.BlockSpec((r, c), lambda i: (0, 0))
    outs = (jax.ShapeDtypeStruct((_NP, _D), jnp.float32),) * 3
    common = [row(_TW), row(1), row(_NV), full(_D, _D), full(_D, _D),
              full(_TW, _D), full(1, _D), full(_NV, _D), full(1, _D)]
    if stats is None:
        return pl.pallas_call(
            _pre_body_raw, grid=(g,),
            in_specs=[row(_D)] + common,
            out_specs=(row(_D),) * 3, out_shape=outs,
        )(y, u, pos, var, wxi, wxj, wdu, wdp, wvar, bm)
    return pl.pallas_call(
        _pre_body_norm, grid=(g,),
        in_specs=[row(_D), full(8, _D)] + common,
        out_specs=(row(_D),) * 3, out_shape=outs,
    )(y, stats, u, pos, var, wxi, wxj, wdu, wdp, wvar, bm)


def _edge_body(mp_ref, w, b, out_ref):
    h = _silu(mp_ref[...])
    out_ref[...] = _silu(_dot(h, w[...]) + b[...])


def _edge_mlp(mp, w, b):
    g = _EP // _EB
    return pl.pallas_call(
        _edge_body, grid=(g,),
        in_specs=[pl.BlockSpec((_EB, _D), lambda i: (i, 0)),
                  pl.BlockSpec((_D, _D), lambda i: (0, 0)),
                  pl.BlockSpec((1, _D), lambda i: (0, 0))],
        out_specs=pl.BlockSpec((_EB, _D), lambda i: (i, 0)),
        out_shape=jax.ShapeDtypeStruct((_EP, _D), jnp.float32),
    )(mp, w, b)


def _upd_body(xn_ref, a0_ref, a1_ref, d0_ref, d1_ref, var_ref,
              wux, wua, wuv, bu1, wu2, bu2, y_out, st_out):
    i = pl.program_id(0)
    deg = jnp.clip(d0_ref[:, 0:1] + d1_ref[:, 0:1], 1.0, None)
    agg = (a0_ref[...] + a1_ref[...]) / deg
    xn = xn_ref[...]
    h = _silu(_dot(xn, wux[...]) + _dot(agg, wua[...])
              + _dot(var_ref[...], wuv[...]) + bu1[...])
    y = xn + _silu(_dot(h, wu2[...]) + bu2[...])
    y_out[...] = y
    rows = i * _NB + lax.broadcasted_iota(jnp.int32, (_NB, 1), 0)
    ym = jnp.where(rows < _N, y, 0.0)
    s = jnp.sum(ym, axis=0, keepdims=True)
    ss = jnp.sum(ym * ym, axis=0, keepdims=True)
    blk = jnp.concatenate([s, ss, jnp.zeros((6, _D), jnp.float32)], axis=0)

    @pl.when(i == 0)
    def _():
        st_out[...] = blk

    @pl.when(i != 0)
    def _():
        st_out[...] = st_out[...] + blk


def _update(xn, a0, a1, d0, d1, var, wux, wua, wuv, bu1, wu2, bu2):
    g = _NP // _NB
    row = lambda c: pl.BlockSpec((_NB, c), lambda i: (i, 0))
    full = lambda r, c: pl.BlockSpec((r, c), lambda i: (0, 0))
    return pl.pallas_call(
        _upd_body, grid=(g,),
        in_specs=[row(_D), row(_D), row(_D), row(8), row(8), row(_NV),
                  full(_D, _D), full(_D, _D), full(_NV, _D), full(1, _D),
                  full(_D, _D), full(1, _D)],
        out_specs=(row(_D), full(8, _D)),
        out_shape=(jax.ShapeDtypeStruct((_NP, _D), jnp.float32),
                   jax.ShapeDtypeStruct((8, _D), jnp.float32)),
    )(xn, a0, a1, d0, d1, var, wux, wua, wuv, bu1, wu2, bu2)


def _dec_body(y_ref, st_ref, u_ref, dt_ref, w1m, b1f, w2m, b2, out_ref):
    mean = st_ref[0:1, :] * (1.0 / _N)
    ex2 = st_ref[1:2, :] * (1.0 / _N)
    rstd = lax.rsqrt(ex2 - mean * mean + 1e-5)
    xn = (y_ref[...] - mean) * rstd
    h1 = _silu(_dot(xn, w1m[...]) + b1f[...])
    diff = _dot(h1, w2m[...]) + b2[0, 0]
    tgrid = lax.broadcasted_iota(jnp.float32, (1, _TW), 1) + 1.0
    dtv = tgrid * dt_ref[0, 0]
    out_ref[...] = u_ref[:, _TW - 1:_TW] + dtv * diff


def _decoder(y, stats, u, dt, w1m, b1f, w2m, b2):
    g = _NP // _NB
    row = lambda c: pl.BlockSpec((_NB, c), lambda i: (i, 0))
    full = lambda r, c: pl.BlockSpec((r, c), lambda i: (0, 0))
    return pl.pallas_call(
        _dec_body, grid=(g,),
        in_specs=[row(_D), full(8, _D), row(_TW), full(1, 1),
                  full(_D, 304), full(1, 304), full(304, _TW), full(1, 1)],
        out_specs=row(_TW),
        out_shape=jax.ShapeDtypeStruct((_NP, _TW), jnp.float32),
    )(y, stats, u, dt, w1m, b1f, w2m, b2)


_MESH = plsc.VectorSubcoreMesh(core_axis_name="c", subcore_axis_name="s")


def _sc_gather(dst2, src2, tab_d, tab_s):
    @functools.partial(
        pl.kernel,
        out_type=jax.ShapeDtypeStruct((_EP, _D), jnp.float32),
        mesh=_MESH,
        scratch_types=[
            pltpu.VMEM((_CPW, _CH), jnp.int32),
            pltpu.VMEM((_CPW, _CH), jnp.int32),
            pltpu.VMEM((_CH, _D), jnp.float32),
            pltpu.VMEM((_CH, _D), jnp.float32),
            pltpu.SemaphoreType.DMA,
            pltpu.SemaphoreType.DMA,
        ],
    )
    def k(dst_hbm, src_hbm, td_hbm, ts_hbm, out_hbm, idxd, idxs, ra, rb, sa, sb):
        cid = lax.axis_index("c")
        sid = lax.axis_index("s")
        wid = sid * _NC + cid
        c0 = wid * _CPW
        pltpu.sync_copy(dst_hbm.at[pl.ds(c0, _CPW)], idxd)
        pltpu.sync_copy(src_hbm.at[pl.ds(c0, _CPW)], idxs)

        def body(j, _):
            ca = pltpu.async_copy(td_hbm.at[idxd.at[j]], ra, sa)
            cb = pltpu.async_copy(ts_hbm.at[idxs.at[j]], rb, sb)
            ca.wait()
            cb.wait()

            def add_row(r, _):
                for q in range(_D // 16):
                    sl = pl.ds(q * 16, 16)
                    ra[r, sl] = ra[r, sl] + rb[r, sl]
                return 0

            lax.fori_loop(0, _CH, add_row, 0)
            pltpu.sync_copy(ra, out_hbm.at[pl.ds((c0 + j) * _CH, _CH)])
            return 0

        lax.fori_loop(0, _CPW, body, 0)

    return k(dst2, src2, tab_d, tab_s)


def _sc_scatter(m2, dst2, with_deg):
    out_types = [jax.ShapeDtypeStruct((_NC, _NP, _D), jnp.float32)]
    scratch = [
        pltpu.VMEM((_CPW, _CH), jnp.int32),
        pltpu.VMEM((_CH, _D), jnp.float32),
        pltpu.VMEM_SHARED((_NP, _D), jnp.float32),
    ]
    if with_deg:
        out_types.append(jax.ShapeDtypeStruct((_NC, _NP, 8), jnp.float32))
        scratch += [
            pltpu.VMEM((_CH, 8), jnp.float32),
            pltpu.VMEM((_CH, 8), jnp.float32),
            pltpu.VMEM_SHARED((_NP, 8), jnp.float32),
        ]

    def zero_vmem(ref, rows, cols):
        z = jnp.zeros((16,), jnp.float32)

        def zr(r, _):
            for q in range(cols // 16):
                ref[r, pl.ds(q * 16, 16)] = z
            return 0

        lax.fori_loop(0, rows, zr, 0)

    def common(m2_hbm, dst_hbm, agg_hbm, idxb, rows, acc,
               ones=None, zb8=None, dacc=None, deg_hbm=None):
        cid = lax.axis_index("c")
        sid = lax.axis_index("s")
        wid = sid * _NC + cid
        c0 = wid * _CPW
        r0 = sid * _RPT
        pltpu.sync_copy(dst_hbm.at[pl.ds(c0, _CPW)], idxb)
        zero_vmem(rows, _CH, _D)
        for t in range(_RPT // _CH):
            pltpu.sync_copy(rows, acc.at[pl.ds(r0 + t * _CH, _CH)])
        if dacc is not None:
            zero_vmem(zb8, _CH, 8)

            def fill_ones(r, _):
                ones[r, pl.ds(0, 16)] = jnp.full((16,), 1.0, jnp.float32)
                return 0

            lax.fori_loop(0, _CH * 8 // 16, fill_ones, 0)
            for t in range(_RPT // _CH):
                pltpu.sync_copy(zb8, dacc.at[pl.ds(r0 + t * _CH, _CH)])
        plsc.subcore_barrier()

        def body(j, _):
            pltpu.sync_copy(m2_hbm.at[pl.ds((c0 + j) * _CH, _CH)], rows)
            pltpu.sync_copy(rows, acc.at[idxb.at[j]], add=True)
            if dacc is not None:
                pltpu.sync_copy(ones, dacc.at[idxb.at[j]], add=True)
            return 0

        lax.fori_loop(0, _CPW, body, 0)
        plsc.subcore_barrier()
        pltpu.sync_copy(acc.at[pl.ds(r0, _RPT)], agg_hbm.at[cid, pl.ds(r0, _RPT)])
        if dacc is not None:
            pltpu.sync_copy(dacc.at[pl.ds(r0, _RPT)], deg_hbm.at[cid, pl.ds(r0, _RPT)])

    if with_deg:
        @functools.partial(pl.kernel, out_type=tuple(out_types), mesh=_MESH,
                           scratch_types=scratch)
        def kd(m2_hbm, dst_hbm, agg_hbm, deg_hbm, idxb, rows, acc, ones, zb8, dacc):
            common(m2_hbm, dst_hbm, agg_hbm, idxb, rows, acc,
                   ones=ones, zb8=zb8, dacc=dacc, deg_hbm=deg_hbm)

        return kd(m2, dst2)

    @functools.partial(pl.kernel, out_type=tuple(out_types), mesh=_MESH,
                       scratch_types=scratch)
    def k(m2_hbm, dst_hbm, agg_hbm, idxb, rows, acc):
        common(m2_hbm, dst_hbm, agg_hbm, idxb, rows, acc)

    return (k(m2, dst2)[0], None)


_M1 = (np.arange(128)[:, None, None]
       == 3 * np.arange(38)[None, :, None] + np.arange(16)[None, None, :]
       ).astype(np.float32)
_M2 = (np.arange(38)[:, None, None]
       == np.arange(25)[None, :, None] + np.arange(14)[None, None, :]
       ).astype(np.float32)


def kernel(u, pos, variables, edge_index, dt, enc_W1, enc_b1, enc_W2, enc_b2,
           Wm1, bm1, Wm2, bm2, Wu1, bu1, Wu2, bu2, dec_W1, dec_b1, dec_W2, dec_b2):
    padn = _NP - _N
    u_p = jnp.pad(u, ((0, padn), (0, 0)))
    pos_p = jnp.pad(pos, ((0, padn), (0, 0)))
    var_p = jnp.pad(variables, ((0, padn), (0, 0)))

    src = edge_index[0].astype(jnp.int32)
    dst = edge_index[1].astype(jnp.int32)
    pade = _EP - _E
    fillv = jnp.full((pade,), _DUMP, jnp.int32)
    src2 = jnp.concatenate([src, fillv]).reshape(_EP // _CH, _CH)
    dst2 = jnp.concatenate([dst, fillv]).reshape(_EP // _CH, _CH)

    w1u, w1p, w1v = enc_W1[:_TW], enc_W1[_TW:_TW + 1], enc_W1[_TW + 1:]
    b1 = enc_b1.reshape(1, _D)
    b2 = enc_b2.reshape(1, _D)

    x = _encoder(u_p, pos_p, var_p, w1u, w1p, w1v, b1, enc_W2, b2)

    w1m = jnp.einsum('hjk,ok->hoj', jnp.asarray(_M1), dec_W1[:, 0, :]).reshape(_D, 304)
    b1f = jnp.repeat(dec_b1, 38).reshape(1, 304)
    w2m = jnp.einsum('pjk,ok->opj', jnp.asarray(_M2), dec_W2[0]).reshape(304, _TW)

    stats = None
    d0 = d1 = None
    y = x
    for i in range(_L):
        W = Wm1[i]
        wxi, wxj = W[:_D], W[_D:2 * _D]
        wdu = W[2 * _D:2 * _D + _TW]
        wdp = W[2 * _D + _TW:2 * _D + _TW + 1]
        wvar = W[2 * _D + _TW + 1:]
        bm = bm1[i].reshape(1, _D)
        xn, ad, asrc = _pre(y, stats, u_p, pos_p, var_p,
                            wxi, wxj, wdu, wdp, wvar, bm)
        mpre = _sc_gather(dst2, src2, ad, asrc)
        m2 = _edge_mlp(mpre, Wm2[i], bm2[i].reshape(1, _D))
        aggp, degp = _sc_scatter(m2, dst2, with_deg=(i == 0))
        if i == 0:
            d0, d1 = degp[0], degp[1]
        Wu = Wu1[i]
        y, stats = _update(xn, aggp[0], aggp[1], d0, d1, var_p,
                           Wu[:_D], Wu[_D:2 * _D], Wu[2 * _D:],
                           bu1[i].reshape(1, _D), Wu2[i], bu2[i].reshape(1, _D))

    out = _decoder(y, stats, u_p, dt.reshape(1, 1), w1m, b1f, w2m,
                   dec_b2.reshape(1, 1))
    return out[:_N][..., None]


# speedup vs baseline: 3.3418x; 3.3418x over previous
import functools

import jax
import jax.numpy as jnp
import numpy as np
from jax import lax
from jax.experimental import pallas as pl
from jax.experimental.pallas import tpu as pltpu
from jax.experimental.pallas import tpu_sc as plsc

_N = 10000
_E = 160000
_TW = 25
_NV = 2
_D = 128
_L = 4

_NP = 10240
_EP = 163840
_NB = 1280
_EB = 2048
_CH = 128
_NC = 2
_NS = 16
_NWORK = _NC * _NS
_CPW = _EP // (_NWORK * _CH)
_RPT = _NP // _NS
_DUMP = _N

_HIGH = lax.Precision.HIGHEST


def _silu(x):
    return x * (1.0 / (1.0 + jnp.exp(-x)))


def _dot(a, b):
    return lax.dot_general(a, b, (((1,), (0,)), ((), ())),
                           precision=_HIGH, preferred_element_type=jnp.float32)


def _enc_body(u_ref, pos_ref, var_ref, w1u, w1p, w1v, b1, w2, b2, x_out):
    h = _dot(u_ref[...], w1u[...]) + _dot(pos_ref[...], w1p[...]) \
        + _dot(var_ref[...], w1v[...]) + b1[...]
    h = _silu(h)
    x_out[...] = _silu(_dot(h, w2[...]) + b2[...])


def _encoder(u, pos, var, w1u, w1p, w1v, b1, w2, b2):
    g = _NP // _NB
    row = lambda c: pl.BlockSpec((_NB, c), lambda i: (i, 0))
    full = lambda r, c: pl.BlockSpec((r, c), lambda i: (0, 0))
    return pl.pallas_call(
        _enc_body,
        grid=(g,),
        in_specs=[row(_TW), row(1), row(_NV), full(_TW, _D), full(1, _D),
                  full(_NV, _D), full(1, _D), full(_D, _D), full(1, _D)],
        out_specs=row(_D),
        out_shape=jax.ShapeDtypeStruct((_NP, _D), jnp.float32),
    )(u, pos, var, w1u, w1p, w1v, b1, w2, b2)


def _pre_body_norm(y_ref, st_ref, u_ref, pos_ref, var_ref,
                   wxi, wxj, wdu, wdp, wvar, bm, xn_out, ad_out, as_out):
    mean = st_ref[0:1, :] * (1.0 / _N)
    ex2 = st_ref[1:2, :] * (1.0 / _N)
    rstd = lax.rsqrt(ex2 - mean * mean + 1e-5)
    xn = (y_ref[...] - mean) * rstd
    t = _dot(u_ref[...], wdu[...]) + _dot(pos_ref[...], wdp[...])
    xn_out[...] = xn
    ad_out[...] = _dot(xn, wxi[...]) + t + _dot(var_ref[...], wvar[...]) + bm[...]
    as_out[...] = _dot(xn, wxj[...]) - t


def _pre_body_raw(y_ref, u_ref, pos_ref, var_ref,
                  wxi, wxj, wdu, wdp, wvar, bm, xn_out, ad_out, as_out):
    xn = y_ref[...]
    t = _dot(u_ref[...], wdu[...]) + _dot(pos_ref[...], wdp[...])
    xn_out[...] = xn
    ad_out[...] = _dot(xn, wxi[...]) + t + _dot(var_ref[...], wvar[...]) + bm[...]
    as_out[...] = _dot(xn, wxj[...]) - t


def _pre(y, stats, u, pos, var, wxi, wxj, wdu, wdp, wvar, bm):
    g = _NP // _NB
    row = lambda c: pl.BlockSpec((_NB, c), lambda i: (i, 0))
    full = lambda r, c: pl.BlockSpec((r, c), lambda i: (0, 0))
    outs = (jax.ShapeDtypeStruct((_NP, _D), jnp.float32),) * 3
    common = [row(_TW), row(1), row(_NV), full(_D, _D), full(_D, _D),
              full(_TW, _D), full(1, _D), full(_NV, _D), full(1, _D)]
    if stats is None:
        return pl.pallas_call(
            _pre_body_raw, grid=(g,),
            in_specs=[row(_D)] + common,
            out_specs=(row(_D),) * 3, out_shape=outs,
        )(y, u, pos, var, wxi, wxj, wdu, wdp, wvar, bm)
    return pl.pallas_call(
        _pre_body_norm, grid=(g,),
        in_specs=[row(_D), full(8, _D)] + common,
        out_specs=(row(_D),) * 3, out_shape=outs,
    )(y, stats, u, pos, var, wxi, wxj, wdu, wdp, wvar, bm)


def _edge_body(mp_ref, w, b, out_ref):
    h = _silu(mp_ref[...])
    out_ref[...] = _silu(_dot(h, w[...]) + b[...])


def _edge_mlp(mp, w, b):
    g = _EP // _EB
    return pl.pallas_call(
        _edge_body, grid=(g,),
        in_specs=[pl.BlockSpec((_EB, _D), lambda i: (i, 0)),
                  pl.BlockSpec((_D, _D), lambda i: (0, 0)),
                  pl.BlockSpec((1, _D), lambda i: (0, 0))],
        out_specs=pl.BlockSpec((_EB, _D), lambda i: (i, 0)),
        out_shape=jax.ShapeDtypeStruct((_EP, _D), jnp.float32),
    )(mp, w, b)


def _upd_body(xn_ref, a0_ref, a1_ref, d0_ref, d1_ref, var_ref,
              wux, wua, wuv, bu1, wu2, bu2, y_out, st_out):
    i = pl.program_id(0)
    deg = jnp.clip(d0_ref[:, 0:1] + d1_ref[:, 0:1], 1.0, None)
    agg = (a0_ref[...] + a1_ref[...]) / deg
    xn = xn_ref[...]
    h = _silu(_dot(xn, wux[...]) + _dot(agg, wua[...])
              + _dot(var_ref[...], wuv[...]) + bu1[...])
    y = xn + _silu(_dot(h, wu2[...]) + bu2[...])
    y_out[...] = y
    rows = i * _NB + lax.broadcasted_iota(jnp.int32, (_NB, 1), 0)
    ym = jnp.where(rows < _N, y, 0.0)
    s = jnp.sum(ym, axis=0, keepdims=True)
    ss = jnp.sum(ym * ym, axis=0, keepdims=True)
    blk = jnp.concatenate([s, ss, jnp.zeros((6, _D), jnp.float32)], axis=0)

    @pl.when(i == 0)
    def _():
        st_out[...] = blk

    @pl.when(i != 0)
    def _():
        st_out[...] = st_out[...] + blk


def _update(xn, a0, a1, d0, d1, var, wux, wua, wuv, bu1, wu2, bu2):
    g = _NP // _NB
    row = lambda c: pl.BlockSpec((_NB, c), lambda i: (i, 0))
    full = lambda r, c: pl.BlockSpec((r, c), lambda i: (0, 0))
    return pl.pallas_call(
        _upd_body, grid=(g,),
        in_specs=[row(_D), row(_D), row(_D), row(8), row(8), row(_NV),
                  full(_D, _D), full(_D, _D), full(_NV, _D), full(1, _D),
                  full(_D, _D), full(1, _D)],
        out_specs=(row(_D), full(8, _D)),
        out_shape=(jax.ShapeDtypeStruct((_NP, _D), jnp.float32),
                   jax.ShapeDtypeStruct((8, _D), jnp.float32)),
    )(xn, a0, a1, d0, d1, var, wux, wua, wuv, bu1, wu2, bu2)


def _dec_body(y_ref, st_ref, u_ref, dt_ref, w1m, b1f, w2m, b2, out_ref):
    mean = st_ref[0:1, :] * (1.0 / _N)
    ex2 = st_ref[1:2, :] * (1.0 / _N)
    rstd = lax.rsqrt(ex2 - mean * mean + 1e-5)
    xn = (y_ref[...] - mean) * rstd
    h1 = _silu(_dot(xn, w1m[...]) + b1f[...])
    diff = _dot(h1, w2m[...]) + b2[0, 0]
    tgrid = lax.broadcasted_iota(jnp.int32, (1, _TW), 1).astype(jnp.float32) + 1.0
    dtv = tgrid * dt_ref[0, 0]
    out_ref[...] = u_ref[:, _TW - 1:_TW] + dtv * diff


def _decoder(y, stats, u, dt, w1m, b1f, w2m, b2):
    g = _NP // _NB
    row = lambda c: pl.BlockSpec((_NB, c), lambda i: (i, 0))
    full = lambda r, c: pl.BlockSpec((r, c), lambda i: (0, 0))
    return pl.pallas_call(
        _dec_body, grid=(g,),
        in_specs=[row(_D), full(8, _D), row(_TW), full(1, 1),
                  full(_D, 304), full(1, 304), full(304, _TW), full(1, 1)],
        out_specs=row(_TW),
        out_shape=jax.ShapeDtypeStruct((_NP, _TW), jnp.float32),
    )(y, stats, u, dt, w1m, b1f, w2m, b2)


_MESH = plsc.VectorSubcoreMesh(core_axis_name="c", subcore_axis_name="s")


def _sc_gather(dst2, src2, tab_d, tab_s):
    @functools.partial(
        pl.kernel,
        out_type=jax.ShapeDtypeStruct((_EP, _D), jnp.float32),
        mesh=_MESH,
        scratch_types=[
            pltpu.VMEM((_CPW, _CH), jnp.int32),
            pltpu.VMEM((_CPW, _CH), jnp.int32),
            pltpu.VMEM((_CH, _D), jnp.float32),
            pltpu.VMEM((_CH, _D), jnp.float32),
            pltpu.SemaphoreType.DMA,
            pltpu.SemaphoreType.DMA,
        ],
    )
    def k(dst_hbm, src_hbm, td_hbm, ts_hbm, out_hbm, idxd, idxs, ra, rb, sa, sb):
        cid = lax.axis_index("c")
        sid = lax.axis_index("s")
        wid = sid * _NC + cid
        c0 = wid * _CPW
        pltpu.sync_copy(dst_hbm.at[pl.ds(c0, _CPW)], idxd)
        pltpu.sync_copy(src_hbm.at[pl.ds(c0, _CPW)], idxs)

        def body(j, _):
            ca = pltpu.async_copy(td_hbm.at[idxd.at[j]], ra, sa)
            cb = pltpu.async_copy(ts_hbm.at[idxs.at[j]], rb, sb)
            ca.wait()
            cb.wait()

            def add_row(r, _):
                for q in range(_D // 16):
                    sl = pl.ds(q * 16, 16)
                    ra[r, sl] = ra[r, sl] + rb[r, sl]
                return 0

            lax.fori_loop(0, _CH, add_row, 0)
            pltpu.sync_copy(ra, out_hbm.at[pl.ds((c0 + j) * _CH, _CH)])
            return 0

        lax.fori_loop(0, _CPW, body, 0)

    return k(dst2, src2, tab_d, tab_s)


def _zero_vmem(ref, rows, cols):
    z = jnp.zeros((16,), jnp.float32)

    def zr(r, _):
        for q in range(cols // 16):
            ref[r, pl.ds(q * 16, 16)] = z
        return 0

    lax.fori_loop(0, rows, zr, 0)


def _sc_scatter(m2, dst2):
    @functools.partial(
        pl.kernel,
        out_type=jax.ShapeDtypeStruct((_NC, _NP, _D), jnp.float32),
        mesh=_MESH,
        scratch_types=[
            pltpu.VMEM((_CPW, _CH), jnp.int32),
            pltpu.VMEM((_CH, _D), jnp.float32),
            pltpu.VMEM_SHARED((_NP, _D), jnp.float32),
        ],
    )
    def k(m2_hbm, dst_hbm, agg_hbm, idxb, rows, acc):
        cid = lax.axis_index("c")
        sid = lax.axis_index("s")
        wid = sid * _NC + cid
        c0 = wid * _CPW
        r0 = sid * _RPT
        pltpu.sync_copy(dst_hbm.at[pl.ds(c0, _CPW)], idxb)
        _zero_vmem(rows, _CH, _D)
        for t in range(_RPT // _CH):
            pltpu.sync_copy(rows, acc.at[pl.ds(r0 + t * _CH, _CH)])
        plsc.subcore_barrier()

        def body(j, _):
            pltpu.sync_copy(m2_hbm.at[pl.ds((c0 + j) * _CH, _CH)], rows)
            pltpu.sync_copy(rows, acc.at[idxb.at[j]], add=True)
            return 0

        lax.fori_loop(0, _CPW, body, 0)
        plsc.subcore_barrier()
        pltpu.sync_copy(acc.at[pl.ds(r0, _RPT)], agg_hbm.at[cid, pl.ds(r0, _RPT)])

    return k(m2, dst2)


def _sc_degree(dst2, ones_c, zeros_c):
    @functools.partial(
        pl.kernel,
        out_type=jax.ShapeDtypeStruct((_NC, _NP, 8), jnp.float32),
        mesh=_MESH,
        scratch_types=[
            pltpu.VMEM((_CPW, _CH), jnp.int32),
            pltpu.VMEM((_CH, 8), jnp.float32),
            pltpu.VMEM_SHARED((_NP, 8), jnp.float32),
        ],
    )
    def k(dst_hbm, ones_hbm, zeros_hbm, deg_hbm, idxb, ones, dacc):
        cid = lax.axis_index("c")
        sid = lax.axis_index("s")
        wid = sid * _NC + cid
        c0 = wid * _CPW
        r0 = sid * _RPT
        pltpu.sync_copy(dst_hbm.at[pl.ds(c0, _CPW)], idxb)
        pltpu.sync_copy(ones_hbm, ones)
        pltpu.sync_copy(zeros_hbm, dacc.at[pl.ds(r0, _RPT)])
        plsc.subcore_barrier()

        def body(j, _):
            pltpu.sync_copy(ones, dacc.at[idxb.at[j]], add=True)
            return 0

        lax.fori_loop(0, _CPW, body, 0)
        plsc.subcore_barrier()
        pltpu.sync_copy(dacc.at[pl.ds(r0, _RPT)], deg_hbm.at[cid, pl.ds(r0, _RPT)])

    return k(dst2, ones_c, zeros_c)


_M1 = (np.arange(128)[:, None, None]
       == 3 * np.arange(38)[None, :, None] + np.arange(16)[None, None, :]
       ).astype(np.float32)
_M2 = (np.arange(38)[:, None, None]
       == np.arange(25)[None, :, None] + np.arange(14)[None, None, :]
       ).astype(np.float32)


def kernel(u, pos, variables, edge_index, dt, enc_W1, enc_b1, enc_W2, enc_b2,
           Wm1, bm1, Wm2, bm2, Wu1, bu1, Wu2, bu2, dec_W1, dec_b1, dec_W2, dec_b2):
    padn = _NP - _N
    u_p = jnp.pad(u, ((0, padn), (0, 0)))
    pos_p = jnp.pad(pos, ((0, padn), (0, 0)))
    var_p = jnp.pad(variables, ((0, padn), (0, 0)))

    src = edge_index[0].astype(jnp.int32)
    dst = edge_index[1].astype(jnp.int32)
    pade = _EP - _E
    fillv = jnp.full((pade,), _DUMP, jnp.int32)
    src2 = jnp.concatenate([src, fillv]).reshape(_EP // _CH, _CH)
    dst2 = jnp.concatenate([dst, fillv]).reshape(_EP // _CH, _CH)

    w1u, w1p, w1v = enc_W1[:_TW], enc_W1[_TW:_TW + 1], enc_W1[_TW + 1:]
    b1 = enc_b1.reshape(1, _D)
    b2 = enc_b2.reshape(1, _D)

    x = _encoder(u_p, pos_p, var_p, w1u, w1p, w1v, b1, enc_W2, b2)

    w1m = jnp.einsum('hjk,ok->hoj', jnp.asarray(_M1), dec_W1[:, 0, :]).reshape(_D, 304)
    b1f = jnp.repeat(dec_b1, 38).reshape(1, 304)
    w2m = jnp.einsum('pjk,ok->opj', jnp.asarray(_M2), dec_W2[0]).reshape(304, _TW)

    stats = None
    d0 = d1 = None
    y = x
    for i in range(_L):
        W = Wm1[i]
        wxi, wxj = W[:_D], W[_D:2 * _D]
        wdu = W[2 * _D:2 * _D + _TW]
        wdp = W[2 * _D + _TW:2 * _D + _TW + 1]
        wvar = W[2 * _D + _TW + 1:]
        bm = bm1[i].reshape(1, _D)
        xn, ad, asrc = _pre(y, stats, u_p, pos_p, var_p,
                            wxi, wxj, wdu, wdp, wvar, bm)
        mpre = _sc_gather(dst2, src2, ad, asrc)
        m2 = _edge_mlp(mpre, Wm2[i], bm2[i].reshape(1, _D))
        aggp = _sc_scatter(m2, dst2)
        if i == 0:
            degp = _sc_degree(dst2, jnp.ones((_CH, 8), jnp.float32),
                              jnp.zeros((_RPT, 8), jnp.float32))
            d0, d1 = degp[0], degp[1]
        Wu = Wu1[i]
        y, stats = _update(xn, aggp[0], aggp[1], d0, d1, var_p,
                           Wu[:_D], Wu[_D:2 * _D], Wu[2 * _D:],
                           bu1[i].reshape(1, _D), Wu2[i], bu2[i].reshape(1, _D))

    out = _decoder(y, stats, u_p, dt.reshape(1, 1), w1m, b1f, w2m,
                   dec_b2.reshape(1, 1))
    return out[:_N][..., None]


# speedup vs baseline: 3.9248x; 1.1745x over previous
import functools

import jax
import jax.numpy as jnp
import numpy as np
from jax import lax
from jax.experimental import pallas as pl
from jax.experimental.pallas import tpu as pltpu
from jax.experimental.pallas import tpu_sc as plsc

_N = 10000
_E = 160000
_TW = 25
_NV = 2
_D = 128
_L = 4

_NP = 10240
_EP = 163840
_NB = 1280
_EB = 2048
_CH = 128
_NC = 2
_NS = 16
_NWORK = _NC * _NS
_CPW = _EP // (_NWORK * _CH)
_RPT = _NP // _NS
_DUMP = _N

_HIGH = lax.Precision.HIGHEST


def _silu(x):
    return x * (1.0 / (1.0 + jnp.exp(-x)))


def _dot(a, b):
    return lax.dot_general(a, b, (((1,), (0,)), ((), ())),
                           precision=_HIGH, preferred_element_type=jnp.float32)


def _enc_body(u_ref, pos_ref, var_ref, w1u, w1p, w1v, b1, w2, b2, x_out):
    h = _dot(u_ref[...], w1u[...]) + _dot(pos_ref[...], w1p[...]) \
        + _dot(var_ref[...], w1v[...]) + b1[...]
    h = _silu(h)
    x_out[...] = _silu(_dot(h, w2[...]) + b2[...])


def _encoder(u, pos, var, w1u, w1p, w1v, b1, w2, b2):
    g = _NP // _NB
    row = lambda c: pl.BlockSpec((_NB, c), lambda i: (i, 0))
    full = lambda r, c: pl.BlockSpec((r, c), lambda i: (0, 0))
    return pl.pallas_call(
        _enc_body,
        grid=(g,),
        in_specs=[row(_TW), row(1), row(_NV), full(_TW, _D), full(1, _D),
                  full(_NV, _D), full(1, _D), full(_D, _D), full(1, _D)],
        out_specs=row(_D),
        out_shape=jax.ShapeDtypeStruct((_NP, _D), jnp.float32),
    )(u, pos, var, w1u, w1p, w1v, b1, w2, b2)


def _pre_body_norm(y_ref, st_ref, u_ref, pos_ref, var_ref,
                   wxi, wxj, wdu, wdp, wvar, bm, xn_out, ad_out, as_out):
    mean = st_ref[0:1, :] * (1.0 / _N)
    ex2 = st_ref[1:2, :] * (1.0 / _N)
    rstd = lax.rsqrt(ex2 - mean * mean + 1e-5)
    xn = (y_ref[...] - mean) * rstd
    t = _dot(u_ref[...], wdu[...]) + _dot(pos_ref[...], wdp[...])
    xn_out[...] = xn
    ad_out[...] = _dot(xn, wxi[...]) + t + _dot(var_ref[...], wvar[...]) + bm[...]
    as_out[...] = _dot(xn, wxj[...]) - t


def _pre_body_raw(y_ref, u_ref, pos_ref, var_ref,
                  wxi, wxj, wdu, wdp, wvar, bm, xn_out, ad_out, as_out):
    xn = y_ref[...]
    t = _dot(u_ref[...], wdu[...]) + _dot(pos_ref[...], wdp[...])
    xn_out[...] = xn
    ad_out[...] = _dot(xn, wxi[...]) + t + _dot(var_ref[...], wvar[...]) + bm[...]
    as_out[...] = _dot(xn, wxj[...]) - t


def _pre(y, stats, u, pos, var, wxi, wxj, wdu, wdp, wvar, bm):
    g = _NP // _NB
    row = lambda c: pl.BlockSpec((_NB, c), lambda i: (i, 0))
    full = lambda r, c: pl.BlockSpec((r, c), lambda i: (0, 0))
    outs = (jax.ShapeDtypeStruct((_NP, _D), jnp.float32),) * 3
    common = [row(_TW), row(1), row(_NV), full(_D, _D), full(_D, _D),
              full(_TW, _D), full(1, _D), full(_NV, _D), full(1, _D)]
    if stats is None:
        return pl.pallas_call(
            _pre_body_raw, grid=(g,),
            in_specs=[row(_D)] + common,
            out_specs=(row(_D),) * 3, out_shape=outs,
        )(y, u, pos, var, wxi, wxj, wdu, wdp, wvar, bm)
    return pl.pallas_call(
        _pre_body_norm, grid=(g,),
        in_specs=[row(_D), full(8, _D)] + common,
        out_specs=(row(_D),) * 3, out_shape=outs,
    )(y, stats, u, pos, var, wxi, wxj, wdu, wdp, wvar, bm)


def _edge_body(mp_ref, w, b, out_ref):
    h = _silu(mp_ref[...])
    out_ref[...] = _silu(_dot(h, w[...]) + b[...])


def _edge_mlp(mp, w, b):
    g = _EP // _EB
    return pl.pallas_call(
        _edge_body, grid=(g,),
        in_specs=[pl.BlockSpec((_EB, _D), lambda i: (i, 0)),
                  pl.BlockSpec((_D, _D), lambda i: (0, 0)),
                  pl.BlockSpec((1, _D), lambda i: (0, 0))],
        out_specs=pl.BlockSpec((_EB, _D), lambda i: (i, 0)),
        out_shape=jax.ShapeDtypeStruct((_EP, _D), jnp.float32),
    )(mp, w, b)


def _upd_body(xn_ref, a0_ref, a1_ref, d0_ref, d1_ref, var_ref,
              wux, wua, wuv, bu1, wu2, bu2, y_out, st_out):
    i = pl.program_id(0)
    deg = jnp.clip(d0_ref[:, 0:1] + d1_ref[:, 0:1], 1.0, None)
    agg = (a0_ref[...] + a1_ref[...]) / deg
    xn = xn_ref[...]
    h = _silu(_dot(xn, wux[...]) + _dot(agg, wua[...])
              + _dot(var_ref[...], wuv[...]) + bu1[...])
    y = xn + _silu(_dot(h, wu2[...]) + bu2[...])
    y_out[...] = y
    rows = i * _NB + lax.broadcasted_iota(jnp.int32, (_NB, 1), 0)
    ym = jnp.where(rows < _N, y, 0.0)
    s = jnp.sum(ym, axis=0, keepdims=True)
    ss = jnp.sum(ym * ym, axis=0, keepdims=True)
    blk = jnp.concatenate([s, ss, jnp.zeros((6, _D), jnp.float32)], axis=0)

    @pl.when(i == 0)
    def _():
        st_out[...] = blk

    @pl.when(i != 0)
    def _():
        st_out[...] = st_out[...] + blk


def _update(xn, a0, a1, d0, d1, var, wux, wua, wuv, bu1, wu2, bu2):
    g = _NP // _NB
    row = lambda c: pl.BlockSpec((_NB, c), lambda i: (i, 0))
    full = lambda r, c: pl.BlockSpec((r, c), lambda i: (0, 0))
    return pl.pallas_call(
        _upd_body, grid=(g,),
        in_specs=[row(_D), row(_D), row(_D), row(8), row(8), row(_NV),
                  full(_D, _D), full(_D, _D), full(_NV, _D), full(1, _D),
                  full(_D, _D), full(1, _D)],
        out_specs=(row(_D), full(8, _D)),
        out_shape=(jax.ShapeDtypeStruct((_NP, _D), jnp.float32),
                   jax.ShapeDtypeStruct((8, _D), jnp.float32)),
    )(xn, a0, a1, d0, d1, var, wux, wua, wuv, bu1, wu2, bu2)


def _dec_body(y_ref, st_ref, u_ref, dt_ref, w1m, b1f, w2m, b2, out_ref):
    mean = st_ref[0:1, :] * (1.0 / _N)
    ex2 = st_ref[1:2, :] * (1.0 / _N)
    rstd = lax.rsqrt(ex2 - mean * mean + 1e-5)
    xn = (y_ref[...] - mean) * rstd
    h1 = _silu(_dot(xn, w1m[...]) + b1f[...])
    diff = _dot(h1, w2m[...]) + b2[0, 0]
    tgrid = lax.broadcasted_iota(jnp.int32, (1, _TW), 1).astype(jnp.float32) + 1.0
    dtv = tgrid * dt_ref[0, 0]
    out_ref[...] = u_ref[:, _TW - 1:_TW] + dtv * diff


def _decoder(y, stats, u, dt, w1m, b1f, w2m, b2):
    g = _NP // _NB
    row = lambda c: pl.BlockSpec((_NB, c), lambda i: (i, 0))
    full = lambda r, c: pl.BlockSpec((r, c), lambda i: (0, 0))
    return pl.pallas_call(
        _dec_body, grid=(g,),
        in_specs=[row(_D), full(8, _D), row(_TW), full(1, 1),
                  full(_D, 304), full(1, 304), full(304, _TW), full(1, 1)],
        out_specs=row(_TW),
        out_shape=jax.ShapeDtypeStruct((_NP, _TW), jnp.float32),
    )(y, stats, u, dt, w1m, b1f, w2m, b2)


_MESH = plsc.VectorSubcoreMesh(core_axis_name="c", subcore_axis_name="s")


def _sc_gather(dst2, src2, tab_d, tab_s):
    @functools.partial(
        pl.kernel,
        out_type=jax.ShapeDtypeStruct((_EP, _D), jnp.float32),
        mesh=_MESH,
        scratch_types=[
            pltpu.VMEM((_CPW, _CH), jnp.int32),
            pltpu.VMEM((_CPW, _CH), jnp.int32),
            pltpu.VMEM((_CH, _D), jnp.float32),
            pltpu.VMEM((_CH, _D), jnp.float32),
            pltpu.VMEM((_CH, _D), jnp.float32),
            pltpu.VMEM((_CH, _D), jnp.float32),
            pltpu.VMEM((_CH, _D), jnp.float32),
            pltpu.VMEM((_CH, _D), jnp.float32),
            pltpu.SemaphoreType.DMA,
            pltpu.SemaphoreType.DMA,
            pltpu.SemaphoreType.DMA,
            pltpu.SemaphoreType.DMA,
            pltpu.SemaphoreType.DMA,
            pltpu.SemaphoreType.DMA,
        ],
    )
    def k(dst_hbm, src_hbm, td_hbm, ts_hbm, out_hbm, idxd, idxs,
          ra0, ra1, rb0, rb1, ro0, ro1, sa0, sa1, sb0, sb1, sw0, sw1):
        ras, rbs, ros = (ra0, ra1), (rb0, rb1), (ro0, ro1)
        sas, sbs, sws = (sa0, sa1), (sb0, sb1), (sw0, sw1)
        cid = lax.axis_index("c")
        sid = lax.axis_index("s")
        wid = sid * _NC + cid
        c0 = wid * _CPW
        pltpu.sync_copy(dst_hbm.at[pl.ds(c0, _CPW)], idxd)
        pltpu.sync_copy(src_hbm.at[pl.ds(c0, _CPW)], idxs)
        for b in range(2):
            pltpu.async_copy(td_hbm.at[idxd.at[b]], ras[b], sas[b])
            pltpu.async_copy(ts_hbm.at[idxs.at[b]], rbs[b], sbs[b])

        def outer(g, _):
            for b in range(2):
                j = g * 2 + b
                ra, rb, ro = ras[b], rbs[b], ros[b]
                pltpu.make_async_copy(td_hbm.at[pl.ds(0, _CH)], ra, sas[b]).wait()
                pltpu.make_async_copy(ts_hbm.at[pl.ds(0, _CH)], rb, sbs[b]).wait()

                @pl.when(j >= 2)
                def _():
                    pltpu.make_async_copy(out_hbm.at[pl.ds(0, _CH)], ro,
                                          sws[b]).wait()

                def add_row(r, _):
                    for q in range(_D // 16):
                        sl = pl.ds(q * 16, 16)
                        ro[r, sl] = ra[r, sl] + rb[r, sl]
                    return 0

                lax.fori_loop(0, _CH, add_row, 0)
                pltpu.async_copy(ro, out_hbm.at[pl.ds((c0 + j) * _CH, _CH)],
                                 sws[b])

                @pl.when(j + 2 < _CPW)
                def _():
                    pltpu.async_copy(td_hbm.at[idxd.at[j + 2]], ra, sas[b])
                    pltpu.async_copy(ts_hbm.at[idxs.at[j + 2]], rb, sbs[b])

            return 0

        lax.fori_loop(0, _CPW // 2, outer, 0)
        for b in range(2):
            pltpu.make_async_copy(out_hbm.at[pl.ds(0, _CH)], ros[b],
                                  sws[b]).wait()

    return k(dst2, src2, tab_d, tab_s)


def _zero_vmem(ref, rows, cols):
    z = jnp.zeros((16,), jnp.float32)

    def zr(r, _):
        for q in range(cols // 16):
            ref[r, pl.ds(q * 16, 16)] = z
        return 0

    lax.fori_loop(0, rows, zr, 0)


def _sc_scatter(m2, dst2):
    @functools.partial(
        pl.kernel,
        out_type=jax.ShapeDtypeStruct((_NC, _NP, _D), jnp.float32),
        mesh=_MESH,
        scratch_types=[
            pltpu.VMEM((_CPW, _CH), jnp.int32),
            pltpu.VMEM((_CH, _D), jnp.float32),
            pltpu.VMEM((_CH, _D), jnp.float32),
            pltpu.VMEM_SHARED((_NP, _D), jnp.float32),
            pltpu.SemaphoreType.DMA,
            pltpu.SemaphoreType.DMA,
        ],
    )
    def k(m2_hbm, dst_hbm, agg_hbm, idxb, r0buf, r1buf, acc, sl0, sl1):
        rbufs, sls = (r0buf, r1buf), (sl0, sl1)
        cid = lax.axis_index("c")
        sid = lax.axis_index("s")
        wid = sid * _NC + cid
        c0 = wid * _CPW
        r0 = sid * _RPT
        pltpu.sync_copy(dst_hbm.at[pl.ds(c0, _CPW)], idxb)
        _zero_vmem(r0buf, _CH, _D)
        for t in range(_RPT // _CH):
            pltpu.sync_copy(r0buf, acc.at[pl.ds(r0 + t * _CH, _CH)])
        plsc.subcore_barrier()
        for b in range(2):
            pltpu.async_copy(m2_hbm.at[pl.ds((c0 + b) * _CH, _CH)],
                             rbufs[b], sls[b])

        def outer(g, _):
            for b in range(2):
                j = g * 2 + b
                rows = rbufs[b]
                pltpu.make_async_copy(m2_hbm.at[pl.ds(0, _CH)], rows,
                                      sls[b]).wait()
                pltpu.sync_copy(rows, acc.at[idxb.at[j]], add=True)

                @pl.when(j + 2 < _CPW)
                def _():
                    pltpu.async_copy(m2_hbm.at[pl.ds((c0 + j + 2) * _CH, _CH)],
                                     rows, sls[b])

            return 0

        lax.fori_loop(0, _CPW // 2, outer, 0)
        plsc.subcore_barrier()
        pltpu.sync_copy(acc.at[pl.ds(r0, _RPT)], agg_hbm.at[cid, pl.ds(r0, _RPT)])

    return k(m2, dst2)


def _sc_degree(dst2, ones_c, zeros_c):
    @functools.partial(
        pl.kernel,
        out_type=jax.ShapeDtypeStruct((_NC, _NP, 8), jnp.float32),
        mesh=_MESH,
        scratch_types=[
            pltpu.VMEM((_CPW, _CH), jnp.int32),
            pltpu.VMEM((_CH, 8), jnp.float32),
            pltpu.VMEM_SHARED((_NP, 8), jnp.float32),
        ],
    )
    def k(dst_hbm, ones_hbm, zeros_hbm, deg_hbm, idxb, ones, dacc):
        cid = lax.axis_index("c")
        sid = lax.axis_index("s")
        wid = sid * _NC + cid
        c0 = wid * _CPW
        r0 = sid * _RPT
        pltpu.sync_copy(dst_hbm.at[pl.ds(c0, _CPW)], idxb)
        pltpu.sync_copy(ones_hbm, ones)
        pltpu.sync_copy(zeros_hbm, dacc.at[pl.ds(r0, _RPT)])
        plsc.subcore_barrier()

        def body(j, _):
            pltpu.sync_copy(ones, dacc.at[idxb.at[j]], add=True)
            return 0

        lax.fori_loop(0, _CPW, body, 0)
        plsc.subcore_barrier()
        pltpu.sync_copy(dacc.at[pl.ds(r0, _RPT)], deg_hbm.at[cid, pl.ds(r0, _RPT)])

    return k(dst2, ones_c, zeros_c)


_M1 = (np.arange(128)[:, None, None]
       == 3 * np.arange(38)[None, :, None] + np.arange(16)[None, None, :]
       ).astype(np.float32)
_M2 = (np.arange(38)[:, None, None]
       == np.arange(25)[None, :, None] + np.arange(14)[None, None, :]
       ).astype(np.float32)


def kernel(u, pos, variables, edge_index, dt, enc_W1, enc_b1, enc_W2, enc_b2,
           Wm1, bm1, Wm2, bm2, Wu1, bu1, Wu2, bu2, dec_W1, dec_b1, dec_W2, dec_b2):
    padn = _NP - _N
    u_p = jnp.pad(u, ((0, padn), (0, 0)))
    pos_p = jnp.pad(pos, ((0, padn), (0, 0)))
    var_p = jnp.pad(variables, ((0, padn), (0, 0)))

    src = edge_index[0].astype(jnp.int32)
    dst = edge_index[1].astype(jnp.int32)
    pade = _EP - _E
    fillv = jnp.full((pade,), _DUMP, jnp.int32)
    src2 = jnp.concatenate([src, fillv]).reshape(_EP // _CH, _CH)
    dst2 = jnp.concatenate([dst, fillv]).reshape(_EP // _CH, _CH)

    w1u, w1p, w1v = enc_W1[:_TW], enc_W1[_TW:_TW + 1], enc_W1[_TW + 1:]
    b1 = enc_b1.reshape(1, _D)
    b2 = enc_b2.reshape(1, _D)

    x = _encoder(u_p, pos_p, var_p, w1u, w1p, w1v, b1, enc_W2, b2)

    w1m = jnp.einsum('hjk,ok->hoj', jnp.asarray(_M1), dec_W1[:, 0, :]).reshape(_D, 304)
    b1f = jnp.repeat(dec_b1, 38).reshape(1, 304)
    w2m = jnp.einsum('pjk,ok->opj', jnp.asarray(_M2), dec_W2[0]).reshape(304, _TW)

    stats = None
    d0 = d1 = None
    y = x
    for i in range(_L):
        W = Wm1[i]
        wxi, wxj = W[:_D], W[_D:2 * _D]
        wdu = W[2 * _D:2 * _D + _TW]
        wdp = W[2 * _D + _TW:2 * _D + _TW + 1]
        wvar = W[2 * _D + _TW + 1:]
        bm = bm1[i].reshape(1, _D)
        xn, ad, asrc = _pre(y, stats, u_p, pos_p, var_p,
                            wxi, wxj, wdu, wdp, wvar, bm)
        mpre = _sc_gather(dst2, src2, ad, asrc)
        m2 = _edge_mlp(mpre, Wm2[i], bm2[i].reshape(1, _D))
        aggp = _sc_scatter(m2, dst2)
        if i == 0:
            degp = _sc_degree(dst2, jnp.ones((_CH, 8), jnp.float32),
                              jnp.zeros((_RPT, 8), jnp.float32))
            d0, d1 = degp[0], degp[1]
        Wu = Wu1[i]
        y, stats = _update(xn, aggp[0], aggp[1], d0, d1, var_p,
                           Wu[:_D], Wu[_D:2 * _D], Wu[2 * _D:],
                           bu1[i].reshape(1, _D), Wu2[i], bu2[i].reshape(1, _D))

    out = _decoder(y, stats, u_p, dt.reshape(1, 1), w1m, b1f, w2m,
                   dec_b2.reshape(1, 1))
    return out[:_N][..., None]


# speedup vs baseline: 6.2641x; 1.5960x over previous
import functools

import jax
import jax.numpy as jnp
import numpy as np
from jax import lax
from jax.experimental import pallas as pl
from jax.experimental.pallas import tpu as pltpu
from jax.experimental.pallas import tpu_sc as plsc

_N = 10000
_E = 160000
_TW = 25
_NV = 2
_D = 128
_L = 4

_NP = 10240
_EP = 163840
_NB = 1280
_EB = 2048
_CH = 128
_NC = 2
_NS = 16
_NWORK = _NC * _NS
_CPW = _EP // (_NWORK * _CH)
_RPT = _NP // _NS
_DUMP = _N

_HIGH = lax.Precision.HIGHEST


def _silu(x):
    return x * (1.0 / (1.0 + jnp.exp(-x)))


def _dot(a, b):
    return lax.dot_general(a, b, (((1,), (0,)), ((), ())),
                           precision=_HIGH, preferred_element_type=jnp.float32)


def _enc_body(u_ref, pos_ref, var_ref, w1u, w1p, w1v, b1, w2, b2, x_out):
    h = _dot(u_ref[...], w1u[...]) + _dot(pos_ref[...], w1p[...]) \
        + _dot(var_ref[...], w1v[...]) + b1[...]
    h = _silu(h)
    x_out[...] = _silu(_dot(h, w2[...]) + b2[...])


def _encoder(u, pos, var, w1u, w1p, w1v, b1, w2, b2):
    g = _NP // _NB
    row = lambda c: pl.BlockSpec((_NB, c), lambda i: (i, 0))
    full = lambda r, c: pl.BlockSpec((r, c), lambda i: (0, 0))
    return pl.pallas_call(
        _enc_body,
        grid=(g,),
        in_specs=[row(_TW), row(1), row(_NV), full(_TW, _D), full(1, _D),
                  full(_NV, _D), full(1, _D), full(_D, _D), full(1, _D)],
        out_specs=row(_D),
        out_shape=jax.ShapeDtypeStruct((_NP, _D), jnp.float32),
    )(u, pos, var, w1u, w1p, w1v, b1, w2, b2)


def _pre_body_norm(y_ref, st_ref, u_ref, pos_ref, var_ref,
                   wxi, wxj, wdu, wdp, wvar, bm, xn_out, tab_out):
    mean = st_ref[0:1, :] * (1.0 / _N)
    ex2 = st_ref[1:2, :] * (1.0 / _N)
    rstd = lax.rsqrt(ex2 - mean * mean + 1e-5)
    xn = (y_ref[...] - mean) * rstd
    t = _dot(u_ref[...], wdu[...]) + _dot(pos_ref[...], wdp[...])
    xn_out[...] = xn
    tab_out[0] = _dot(xn, wxi[...]) + t + _dot(var_ref[...], wvar[...]) + bm[...]
    tab_out[1] = _dot(xn, wxj[...]) - t


def _pre_body_raw(y_ref, u_ref, pos_ref, var_ref,
                  wxi, wxj, wdu, wdp, wvar, bm, xn_out, tab_out):
    xn = y_ref[...]
    t = _dot(u_ref[...], wdu[...]) + _dot(pos_ref[...], wdp[...])
    xn_out[...] = xn
    tab_out[0] = _dot(xn, wxi[...]) + t + _dot(var_ref[...], wvar[...]) + bm[...]
    tab_out[1] = _dot(xn, wxj[...]) - t


def _pre(y, stats, u, pos, var, wxi, wxj, wdu, wdp, wvar, bm):
    g = _NP // _NB
    row = lambda c: pl.BlockSpec((_NB, c), lambda i: (i, 0))
    full = lambda r, c: pl.BlockSpec((r, c), lambda i: (0, 0))
    outs = (jax.ShapeDtypeStruct((_NP, _D), jnp.float32),
            jax.ShapeDtypeStruct((_NC, _NP, _D), jnp.float32))
    out_specs = (row(_D), pl.BlockSpec((_NC, _NB, _D), lambda i: (0, i, 0)))
    common = [row(_TW), row(1), row(_NV), full(_D, _D), full(_D, _D),
              full(_TW, _D), full(1, _D), full(_NV, _D), full(1, _D)]
    if stats is None:
        return pl.pallas_call(
            _pre_body_raw, grid=(g,),
            in_specs=[row(_D)] + common,
            out_specs=out_specs, out_shape=outs,
        )(y, u, pos, var, wxi, wxj, wdu, wdp, wvar, bm)
    return pl.pallas_call(
        _pre_body_norm, grid=(g,),
        in_specs=[row(_D), full(8, _D)] + common,
        out_specs=out_specs, out_shape=outs,
    )(y, stats, u, pos, var, wxi, wxj, wdu, wdp, wvar, bm)


def _edge_body(ma_ref, mb_ref, w, b, out_ref):
    h = _silu(ma_ref[0] + mb_ref[0])
    out_ref[...] = _silu(_dot(h, w[...]) + b[...])


def _edge_mlp(mpab, w, b):
    g = _EP // _EB
    return pl.pallas_call(
        _edge_body, grid=(g,),
        in_specs=[pl.BlockSpec((1, _EB, _D), lambda i: (0, i, 0)),
                  pl.BlockSpec((1, _EB, _D), lambda i: (1, i, 0)),
                  pl.BlockSpec((_D, _D), lambda i: (0, 0)),
                  pl.BlockSpec((1, _D), lambda i: (0, 0))],
        out_specs=pl.BlockSpec((_EB, _D), lambda i: (i, 0)),
        out_shape=jax.ShapeDtypeStruct((_EP, _D), jnp.float32),
    )(mpab, mpab, w, b)


def _upd_body(xn_ref, a0_ref, a1_ref, d0_ref, d1_ref, var_ref,
              wux, wua, wuv, bu1, wu2, bu2, y_out, st_out):
    i = pl.program_id(0)
    deg = jnp.clip(d0_ref[:, 0:1] + d1_ref[:, 0:1], 1.0, None)
    agg = (a0_ref[...] + a1_ref[...]) / deg
    xn = xn_ref[...]
    h = _silu(_dot(xn, wux[...]) + _dot(agg, wua[...])
              + _dot(var_ref[...], wuv[...]) + bu1[...])
    y = xn + _silu(_dot(h, wu2[...]) + bu2[...])
    y_out[...] = y
    rows = i * _NB + lax.broadcasted_iota(jnp.int32, (_NB, 1), 0)
    ym = jnp.where(rows < _N, y, 0.0)
    s = jnp.sum(ym, axis=0, keepdims=True)
    ss = jnp.sum(ym * ym, axis=0, keepdims=True)
    blk = jnp.concatenate([s, ss, jnp.zeros((6, _D), jnp.float32)], axis=0)

    @pl.when(i == 0)
    def _():
        st_out[...] = blk

    @pl.when(i != 0)
    def _():
        st_out[...] = st_out[...] + blk


def _update(xn, a0, a1, d0, d1, var, wux, wua, wuv, bu1, wu2, bu2):
    g = _NP // _NB
    row = lambda c: pl.BlockSpec((_NB, c), lambda i: (i, 0))
    full = lambda r, c: pl.BlockSpec((r, c), lambda i: (0, 0))
    return pl.pallas_call(
        _upd_body, grid=(g,),
        in_specs=[row(_D), row(_D), row(_D), row(8), row(8), row(_NV),
                  full(_D, _D), full(_D, _D), full(_NV, _D), full(1, _D),
                  full(_D, _D), full(1, _D)],
        out_specs=(row(_D), full(8, _D)),
        out_shape=(jax.ShapeDtypeStruct((_NP, _D), jnp.float32),
                   jax.ShapeDtypeStruct((8, _D), jnp.float32)),
    )(xn, a0, a1, d0, d1, var, wux, wua, wuv, bu1, wu2, bu2)


def _dec_body(y_ref, st_ref, u_ref, dt_ref, w1m, b1f, w2m, b2, out_ref):
    mean = st_ref[0:1, :] * (1.0 / _N)
    ex2 = st_ref[1:2, :] * (1.0 / _N)
    rstd = lax.rsqrt(ex2 - mean * mean + 1e-5)
    xn = (y_ref[...] - mean) * rstd
    h1 = _silu(_dot(xn, w1m[...]) + b1f[...])
    diff = _dot(h1, w2m[...]) + b2[0, 0]
    tgrid = lax.broadcasted_iota(jnp.int32, (1, _TW), 1).astype(jnp.float32) + 1.0
    dtv = tgrid * dt_ref[0, 0]
    out_ref[...] = u_ref[:, _TW - 1:_TW] + dtv * diff


def _decoder(y, stats, u, dt, w1m, b1f, w2m, b2):
    g = _NP // _NB
    row = lambda c: pl.BlockSpec((_NB, c), lambda i: (i, 0))
    full = lambda r, c: pl.BlockSpec((r, c), lambda i: (0, 0))
    return pl.pallas_call(
        _dec_body, grid=(g,),
        in_specs=[row(_D), full(8, _D), row(_TW), full(1, 1),
                  full(_D, 304), full(1, 304), full(304, _TW), full(1, 1)],
        out_specs=row(_TW),
        out_shape=jax.ShapeDtypeStruct((_NP, _TW), jnp.float32),
    )(y, stats, u, dt, w1m, b1f, w2m, b2)


_MESH = plsc.VectorSubcoreMesh(core_axis_name="c", subcore_axis_name="s")


_CPT = _EP // _CH // _NS


def _sc_gather(idx3, tabs):
    @functools.partial(
        pl.kernel,
        out_type=jax.ShapeDtypeStruct((_NC, _EP, _D), jnp.float32),
        mesh=_MESH,
        scratch_types=[
            pltpu.VMEM((_CPT, _CH), jnp.int32),
            pltpu.VMEM((_CH, _D), jnp.float32),
            pltpu.VMEM((_CH, _D), jnp.float32),
            pltpu.VMEM_SHARED((_NP, _D), jnp.float32),
            pltpu.SemaphoreType.DMA,
            pltpu.SemaphoreType.DMA,
        ],
    )
    def k(idx_hbm, tab_hbm, out_hbm, idxb, r0b, r1b, table, sw0, sw1):
        rbufs, sws = (r0b, r1b), (sw0, sw1)
        cid = lax.axis_index("c")
        sid = lax.axis_index("s")
        c0 = sid * _CPT
        pltpu.sync_copy(idx_hbm.at[cid, pl.ds(c0, _CPT)], idxb)
        pltpu.sync_copy(tab_hbm.at[cid, pl.ds(sid * _RPT, _RPT)],
                        table.at[pl.ds(sid * _RPT, _RPT)])
        plsc.subcore_barrier()

        def outer(g, _):
            for b in range(2):
                j = g * 2 + b
                rows = rbufs[b]

                @pl.when(j >= 2)
                def _():
                    pltpu.make_async_copy(out_hbm.at[cid, pl.ds(0, _CH)],
                                          rows, sws[b]).wait()

                pltpu.sync_copy(table.at[idxb.at[j]], rows)
                pltpu.async_copy(
                    rows, out_hbm.at[cid, pl.ds((c0 + j) * _CH, _CH)], sws[b])
            return 0

        lax.fori_loop(0, _CPT // 2, outer, 0)
        for b in range(2):
            pltpu.make_async_copy(out_hbm.at[cid, pl.ds(0, _CH)], rbufs[b],
                                  sws[b]).wait()

    return k(idx3, tabs)


def _zero_vmem(ref, rows, cols):
    z = jnp.zeros((16,), jnp.float32)

    def zr(r, _):
        for q in range(cols // 16):
            ref[r, pl.ds(q * 16, 16)] = z
        return 0

    lax.fori_loop(0, rows, zr, 0)


def _sc_scatter(m2, dst2):
    @functools.partial(
        pl.kernel,
        out_type=jax.ShapeDtypeStruct((_NC, _NP, _D), jnp.float32),
        mesh=_MESH,
        scratch_types=[
            pltpu.VMEM((_CPW, _CH), jnp.int32),
            pltpu.VMEM((_CH, _D), jnp.float32),
            pltpu.VMEM((_CH, _D), jnp.float32),
            pltpu.VMEM_SHARED((_NP, _D), jnp.float32),
            pltpu.SemaphoreType.DMA,
            pltpu.SemaphoreType.DMA,
            pltpu.SemaphoreType.DMA,
            pltpu.SemaphoreType.DMA,
        ],
    )
    def k(m2_hbm, dst_hbm, agg_hbm, idxb, r0buf, r1buf, acc, sl0, sl1, ss0, ss1):
        rbufs, sls, sss = (r0buf, r1buf), (sl0, sl1), (ss0, ss1)
        cid = lax.axis_index("c")
        sid = lax.axis_index("s")
        wid = sid * _NC + cid
        c0 = wid * _CPW
        r0 = sid * _RPT
        pltpu.sync_copy(dst_hbm.at[pl.ds(c0, _CPW)], idxb)
        _zero_vmem(r0buf, _CH, _D)
        for t in range(_RPT // _CH):
            pltpu.sync_copy(r0buf, acc.at[pl.ds(r0 + t * _CH, _CH)])
        plsc.subcore_barrier()
        pltpu.async_copy(m2_hbm.at[pl.ds(c0 * _CH, _CH)], rbufs[0], sls[0])

        def outer(g, _):
            for b in range(2):
                j = g * 2 + b
                ob = 1 - b
                rows = rbufs[b]
                pltpu.make_async_copy(m2_hbm.at[pl.ds(0, _CH)], rows,
                                      sls[b]).wait()
                pltpu.async_copy(rows, acc.at[idxb.at[j]], sss[b], add=True)

                @pl.when(j >= 1)
                def _():
                    pltpu.make_async_copy(acc.at[pl.ds(0, _CH)], rbufs[ob],
                                          sss[ob]).wait()

                @pl.when(j + 1 < _CPW)
                def _():
                    pltpu.async_copy(m2_hbm.at[pl.ds((c0 + j + 1) * _CH, _CH)],
                                     rbufs[ob], sls[ob])

            return 0

        lax.fori_loop(0, _CPW // 2, outer, 0)
        pltpu.make_async_copy(acc.at[pl.ds(0, _CH)], rbufs[(_CPW - 1) % 2],
                              sss[(_CPW - 1) % 2]).wait()
        plsc.subcore_barrier()
        pltpu.sync_copy(acc.at[pl.ds(r0, _RPT)], agg_hbm.at[cid, pl.ds(r0, _RPT)])

    return k(m2, dst2)


def _sc_degree(dst2, ones_c, zeros_c):
    @functools.partial(
        pl.kernel,
        out_type=jax.ShapeDtypeStruct((_NC, _NP, 8), jnp.float32),
        mesh=_MESH,
        scratch_types=[
            pltpu.VMEM((_CPW, _CH), jnp.int32),
            pltpu.VMEM((_CH, 8), jnp.float32),
            pltpu.VMEM_SHARED((_NP, 8), jnp.float32),
        ],
    )
    def k(dst_hbm, ones_hbm, zeros_hbm, deg_hbm, idxb, ones, dacc):
        cid = lax.axis_index("c")
        sid = lax.axis_index("s")
        wid = sid * _NC + cid
        c0 = wid * _CPW
        r0 = sid * _RPT
        pltpu.sync_copy(dst_hbm.at[pl.ds(c0, _CPW)], idxb)
        pltpu.sync_copy(ones_hbm, ones)
        pltpu.sync_copy(zeros_hbm, dacc.at[pl.ds(r0, _RPT)])
        plsc.subcore_barrier()

        def body(j, _):
            pltpu.sync_copy(ones, dacc.at[idxb.at[j]], add=True)
            return 0

        lax.fori_loop(0, _CPW, body, 0)
        plsc.subcore_barrier()
        pltpu.sync_copy(dacc.at[pl.ds(r0, _RPT)], deg_hbm.at[cid, pl.ds(r0, _RPT)])

    return k(dst2, ones_c, zeros_c)


_M1 = (np.arange(128)[:, None, None]
       == 3 * np.arange(38)[None, :, None] + np.arange(16)[None, None, :]
       ).astype(np.float32)
_M2 = (np.arange(38)[:, None, None]
       == np.arange(25)[None, :, None] + np.arange(14)[None, None, :]
       ).astype(np.float32)


def kernel(u, pos, variables, edge_index, dt, enc_W1, enc_b1, enc_W2, enc_b2,
           Wm1, bm1, Wm2, bm2, Wu1, bu1, Wu2, bu2, dec_W1, dec_b1, dec_W2, dec_b2):
    padn = _NP - _N
    u_p = jnp.pad(u, ((0, padn), (0, 0)))
    pos_p = jnp.pad(pos, ((0, padn), (0, 0)))
    var_p = jnp.pad(variables, ((0, padn), (0, 0)))

    src = edge_index[0].astype(jnp.int32)
    dst = edge_index[1].astype(jnp.int32)
    pade = _EP - _E
    fillv = jnp.full((pade,), _DUMP, jnp.int32)
    src2 = jnp.concatenate([src, fillv]).reshape(_EP // _CH, _CH)
    dst2 = jnp.concatenate([dst, fillv]).reshape(_EP // _CH, _CH)
    idx3 = jnp.stack([dst2, src2])

    w1u, w1p, w1v = enc_W1[:_TW], enc_W1[_TW:_TW + 1], enc_W1[_TW + 1:]
    b1 = enc_b1.reshape(1, _D)
    b2 = enc_b2.reshape(1, _D)

    x = _encoder(u_p, pos_p, var_p, w1u, w1p, w1v, b1, enc_W2, b2)

    w1m = jnp.einsum('hjk,ok->hoj', jnp.asarray(_M1), dec_W1[:, 0, :]).reshape(_D, 304)
    b1f = jnp.repeat(dec_b1, 38).reshape(1, 304)
    w2m = jnp.einsum('pjk,ok->opj', jnp.asarray(_M2), dec_W2[0]).reshape(304, _TW)

    stats = None
    d0 = d1 = None
    y = x
    for i in range(_L):
        W = Wm1[i]
        wxi, wxj = W[:_D], W[_D:2 * _D]
        wdu = W[2 * _D:2 * _D + _TW]
        wdp = W[2 * _D + _TW:2 * _D + _TW + 1]
        wvar = W[2 * _D + _TW + 1:]
        bm = bm1[i].reshape(1, _D)
        xn, tabs = _pre(y, stats, u_p, pos_p, var_p,
                        wxi, wxj, wdu, wdp, wvar, bm)
        mpab = _sc_gather(idx3, tabs)
        m2 = _edge_mlp(mpab, Wm2[i], bm2[i].reshape(1, _D))
        aggp = _sc_scatter(m2, dst2)
        if i == 0:
            degp = _sc_degree(dst2, jnp.ones((_CH, 8), jnp.float32),
                              jnp.zeros((_RPT, 8), jnp.float32))
            d0, d1 = degp[0], degp[1]
        Wu = Wu1[i]
        y, stats = _update(xn, aggp[0], aggp[1], d0, d1, var_p,
                           Wu[:_D], Wu[_D:2 * _D], Wu[2 * _D:],
                           bu1[i].reshape(1, _D), Wu2[i], bu2[i].reshape(1, _D))

    out = _decoder(y, stats, u_p, dt.reshape(1, 1), w1m, b1f, w2m,
                   dec_b2.reshape(1, 1))
    return out[:_N][..., None]


# speedup vs baseline: 6.7647x; 1.0799x over previous
import functools

import jax
import jax.numpy as jnp
import numpy as np
from jax import lax
from jax.experimental import pallas as pl
from jax.experimental.pallas import tpu as pltpu
from jax.experimental.pallas import tpu_sc as plsc

_N = 10000
_E = 160000
_TW = 25
_NV = 2
_D = 128
_L = 4

_NP = 10240
_EP = 163840
_NB = 1280
_EB = 2048
_CH = 128
_NC = 2
_NS = 16
_NWORK = _NC * _NS
_CPW = _EP // (_NWORK * _CH)
_RPT = _NP // _NS
_DUMP = _N

_HIGH = lax.Precision.HIGHEST


def _silu(x):
    return x * (1.0 / (1.0 + jnp.exp(-x)))


def _dot(a, b, precision=_HIGH):
    return lax.dot_general(a, b, (((1,), (0,)), ((), ())),
                           precision=precision,
                           preferred_element_type=jnp.float32)


def _enc_body(u_ref, pos_ref, var_ref, w1u, w1p, w1v, b1, w2, b2, x_out):
    h = _dot(u_ref[...], w1u[...]) + _dot(pos_ref[...], w1p[...]) \
        + _dot(var_ref[...], w1v[...]) + b1[...]
    h = _silu(h)
    x_out[...] = _silu(_dot(h, w2[...]) + b2[...])


def _encoder(u, pos, var, w1u, w1p, w1v, b1, w2, b2):
    g = _NP // _NB
    row = lambda c: pl.BlockSpec((_NB, c), lambda i: (i, 0))
    full = lambda r, c: pl.BlockSpec((r, c), lambda i: (0, 0))
    return pl.pallas_call(
        _enc_body,
        grid=(g,),
        in_specs=[row(_TW), row(1), row(_NV), full(_TW, _D), full(1, _D),
                  full(_NV, _D), full(1, _D), full(_D, _D), full(1, _D)],
        out_specs=row(_D),
        out_shape=jax.ShapeDtypeStruct((_NP, _D), jnp.float32),
    )(u, pos, var, w1u, w1p, w1v, b1, w2, b2)


def _pre_body_norm(y_ref, st_ref, u_ref, pos_ref, var_ref,
                   wxi, wxj, wdu, wdp, wvar, bm, xn_out, tab_out):
    mean = st_ref[0:1, :] * (1.0 / _N)
    ex2 = st_ref[1:2, :] * (1.0 / _N)
    rstd = lax.rsqrt(ex2 - mean * mean + 1e-5)
    xn = (y_ref[...] - mean) * rstd
    t = _dot(u_ref[...], wdu[...]) + _dot(pos_ref[...], wdp[...])
    xn_out[...] = xn
    tab_out[0] = _dot(xn, wxi[...]) + t + _dot(var_ref[...], wvar[...]) + bm[...]
    tab_out[1] = _dot(xn, wxj[...]) - t


def _pre_body_raw(y_ref, u_ref, pos_ref, var_ref,
                  wxi, wxj, wdu, wdp, wvar, bm, xn_out, tab_out):
    xn = y_ref[...]
    t = _dot(u_ref[...], wdu[...]) + _dot(pos_ref[...], wdp[...])
    xn_out[...] = xn
    tab_out[0] = _dot(xn, wxi[...]) + t + _dot(var_ref[...], wvar[...]) + bm[...]
    tab_out[1] = _dot(xn, wxj[...]) - t


def _pre(y, stats, u, pos, var, wxi, wxj, wdu, wdp, wvar, bm):
    g = _NP // _NB
    row = lambda c: pl.BlockSpec((_NB, c), lambda i: (i, 0))
    full = lambda r, c: pl.BlockSpec((r, c), lambda i: (0, 0))
    outs = (jax.ShapeDtypeStruct((_NP, _D), jnp.float32),
            jax.ShapeDtypeStruct((_NC, _NP, _D), jnp.float32))
    out_specs = (row(_D), pl.BlockSpec((_NC, _NB, _D), lambda i: (0, i, 0)))
    common = [row(_TW), row(1), row(_NV), full(_D, _D), full(_D, _D),
              full(_TW, _D), full(1, _D), full(_NV, _D), full(1, _D)]
    if stats is None:
        return pl.pallas_call(
            _pre_body_raw, grid=(g,),
            in_specs=[row(_D)] + common,
            out_specs=out_specs, out_shape=outs,
        )(y, u, pos, var, wxi, wxj, wdu, wdp, wvar, bm)
    return pl.pallas_call(
        _pre_body_norm, grid=(g,),
        in_specs=[row(_D), full(8, _D)] + common,
        out_specs=out_specs, out_shape=outs,
    )(y, stats, u, pos, var, wxi, wxj, wdu, wdp, wvar, bm)


def _edge_body(ma_ref, mb_ref, w, b, out_ref):
    h = _silu(ma_ref[0] + mb_ref[0])
    out_ref[...] = _silu(_dot(h, w[...], precision=lax.Precision.DEFAULT) + b[...])


def _edge_mlp(mpab, w, b):
    g = _EP // _EB
    return pl.pallas_call(
        _edge_body, grid=(g,),
        in_specs=[pl.BlockSpec((1, _EB, _D), lambda i: (0, i, 0)),
                  pl.BlockSpec((1, _EB, _D), lambda i: (1, i, 0)),
                  pl.BlockSpec((_D, _D), lambda i: (0, 0)),
                  pl.BlockSpec((1, _D), lambda i: (0, 0))],
        out_specs=pl.BlockSpec((_EB, _D), lambda i: (i, 0)),
        out_shape=jax.ShapeDtypeStruct((_EP, _D), jnp.float32),
    )(mpab, mpab, w, b)


def _upd_body(xn_ref, a0_ref, a1_ref, d0_ref, d1_ref, var_ref,
              wux, wua, wuv, bu1, wu2, bu2, y_out, st_out):
    i = pl.program_id(0)
    deg = jnp.clip(d0_ref[0, :, 0:1] + d1_ref[0, :, 0:1], 1.0, None)
    agg = (a0_ref[0] + a1_ref[0]) / deg
    xn = xn_ref[...]
    h = _silu(_dot(xn, wux[...]) + _dot(agg, wua[...])
              + _dot(var_ref[...], wuv[...]) + bu1[...])
    y = xn + _silu(_dot(h, wu2[...]) + bu2[...])
    y_out[...] = y
    rows = i * _NB + lax.broadcasted_iota(jnp.int32, (_NB, 1), 0)
    ym = jnp.where(rows < _N, y, 0.0)
    s = jnp.sum(ym, axis=0, keepdims=True)
    ss = jnp.sum(ym * ym, axis=0, keepdims=True)
    blk = jnp.concatenate([s, ss, jnp.zeros((6, _D), jnp.float32)], axis=0)

    @pl.when(i == 0)
    def _():
        st_out[...] = blk

    @pl.when(i != 0)
    def _():
        st_out[...] = st_out[...] + blk


def _update(xn, aggp, degp, var, wux, wua, wuv, bu1, wu2, bu2):
    g = _NP // _NB
    row = lambda c: pl.BlockSpec((_NB, c), lambda i: (i, 0))
    full = lambda r, c: pl.BlockSpec((r, c), lambda i: (0, 0))
    part = lambda c, s: pl.BlockSpec((1, _NB, c), lambda i, _s=s: (_s, i, 0))
    return pl.pallas_call(
        _upd_body, grid=(g,),
        in_specs=[row(_D), part(_D, 0), part(_D, 1), part(8, 0), part(8, 1),
                  row(_NV),
                  full(_D, _D), full(_D, _D), full(_NV, _D), full(1, _D),
                  full(_D, _D), full(1, _D)],
        out_specs=(row(_D), full(8, _D)),
        out_shape=(jax.ShapeDtypeStruct((_NP, _D), jnp.float32),
                   jax.ShapeDtypeStruct((8, _D), jnp.float32)),
    )(xn, aggp, aggp, degp, degp, var, wux, wua, wuv, bu1, wu2, bu2)


def _dec_body(y_ref, st_ref, u_ref, dt_ref, w1m, b1f, w2m, b2, out_ref):
    mean = st_ref[0:1, :] * (1.0 / _N)
    ex2 = st_ref[1:2, :] * (1.0 / _N)
    rstd = lax.rsqrt(ex2 - mean * mean + 1e-5)
    xn = (y_ref[...] - mean) * rstd
    h1 = _silu(_dot(xn, w1m[...]) + b1f[...])
    diff = _dot(h1, w2m[...]) + b2[0, 0]
    tgrid = lax.broadcasted_iota(jnp.int32, (1, _TW), 1).astype(jnp.float32) + 1.0
    dtv = tgrid * dt_ref[0, 0]
    out_ref[...] = u_ref[:, _TW - 1:_TW] + dtv * diff


def _decoder(y, stats, u, dt, w1m, b1f, w2m, b2):
    g = _NP // _NB
    row = lambda c: pl.BlockSpec((_NB, c), lambda i: (i, 0))
    full = lambda r, c: pl.BlockSpec((r, c), lambda i: (0, 0))
    return pl.pallas_call(
        _dec_body, grid=(g,),
        in_specs=[row(_D), full(8, _D), row(_TW), full(1, 1),
                  full(_D, 304), full(1, 304), full(304, _TW), full(1, 1)],
        out_specs=row(_TW),
        out_shape=jax.ShapeDtypeStruct((_NP, _TW), jnp.float32),
    )(y, stats, u, dt, w1m, b1f, w2m, b2)


_MESH = plsc.VectorSubcoreMesh(core_axis_name="c", subcore_axis_name="s")


_CPT = _EP // _CH // _NS


def _sc_gather(idx3, tabs):
    @functools.partial(
        pl.kernel,
        out_type=jax.ShapeDtypeStruct((_NC, _EP, _D), jnp.float32),
        mesh=_MESH,
        scratch_types=[
            pltpu.VMEM((_CPT, _CH), jnp.int32),
            pltpu.VMEM((_CH, _D), jnp.float32),
            pltpu.VMEM((_CH, _D), jnp.float32),
            pltpu.VMEM_SHARED((_NP, _D), jnp.float32),
            pltpu.SemaphoreType.DMA,
            pltpu.SemaphoreType.DMA,
        ],
    )
    def k(idx_hbm, tab_hbm, out_hbm, idxb, r0b, r1b, table, sw0, sw1):
        rbufs, sws = (r0b, r1b), (sw0, sw1)
        cid = lax.axis_index("c")
        sid = lax.axis_index("s")
        c0 = sid * _CPT
        pltpu.sync_copy(idx_hbm.at[cid, pl.ds(c0, _CPT)], idxb)
        pltpu.sync_copy(tab_hbm.at[cid, pl.ds(sid * _RPT, _RPT)],
                        table.at[pl.ds(sid * _RPT, _RPT)])
        plsc.subcore_barrier()

        def outer(g, _):
            for b in range(2):
                j = g * 2 + b
                rows = rbufs[b]

                @pl.when(j >= 2)
                def _():
                    pltpu.make_async_copy(out_hbm.at[cid, pl.ds(0, _CH)],
                                          rows, sws[b]).wait()

                pltpu.sync_copy(table.at[idxb.at[j]], rows)
                pltpu.async_copy(
                    rows, out_hbm.at[cid, pl.ds((c0 + j) * _CH, _CH)], sws[b])
            return 0

        lax.fori_loop(0, _CPT // 2, outer, 0)
        for b in range(2):
            pltpu.make_async_copy(out_hbm.at[cid, pl.ds(0, _CH)], rbufs[b],
                                  sws[b]).wait()

    return k(idx3, tabs)


def _zero_vmem(ref, rows, cols):
    z = jnp.zeros((16,), jnp.float32)

    def zr(r, _):
        for q in range(cols // 16):
            ref[r, pl.ds(q * 16, 16)] = z
        return 0

    lax.fori_loop(0, rows, zr, 0)


def _sc_scatter(m2, dst2):
    @functools.partial(
        pl.kernel,
        out_type=jax.ShapeDtypeStruct((_NC, _NP, _D), jnp.float32),
        mesh=_MESH,
        scratch_types=[
            pltpu.VMEM((_CPW, _CH), jnp.int32),
            pltpu.VMEM((_CH, _D), jnp.float32),
            pltpu.VMEM((_CH, _D), jnp.float32),
            pltpu.VMEM_SHARED((_NP, _D), jnp.float32),
            pltpu.SemaphoreType.DMA,
            pltpu.SemaphoreType.DMA,
            pltpu.SemaphoreType.DMA,
            pltpu.SemaphoreType.DMA,
        ],
    )
    def k(m2_hbm, dst_hbm, agg_hbm, idxb, r0buf, r1buf, acc, sl0, sl1, ss0, ss1):
        rbufs, sls, sss = (r0buf, r1buf), (sl0, sl1), (ss0, ss1)
        cid = lax.axis_index("c")
        sid = lax.axis_index("s")
        wid = sid * _NC + cid
        c0 = wid * _CPW
        r0 = sid * _RPT
        pltpu.sync_copy(dst_hbm.at[pl.ds(c0, _CPW)], idxb)
        _zero_vmem(r0buf, _CH, _D)
        for t in range(_RPT // _CH):
            pltpu.sync_copy(r0buf, acc.at[pl.ds(r0 + t * _CH, _CH)])
        plsc.subcore_barrier()
        pltpu.async_copy(m2_hbm.at[pl.ds(c0 * _CH, _CH)], rbufs[0], sls[0])

        def outer(g, _):
            for b in range(2):
                j = g * 2 + b
                ob = 1 - b
                rows = rbufs[b]
                pltpu.make_async_copy(m2_hbm.at[pl.ds(0, _CH)], rows,
                                      sls[b]).wait()
                pltpu.async_copy(rows, acc.at[idxb.at[j]], sss[b], add=True)

                @pl.when(j >= 1)
                def _():
                    pltpu.make_async_copy(acc.at[pl.ds(0, _CH)], rbufs[ob],
                                          sss[ob]).wait()

                @pl.when(j + 1 < _CPW)
                def _():
                    pltpu.async_copy(m2_hbm.at[pl.ds((c0 + j + 1) * _CH, _CH)],
                                     rbufs[ob], sls[ob])

            return 0

        lax.fori_loop(0, _CPW // 2, outer, 0)
        pltpu.make_async_copy(acc.at[pl.ds(0, _CH)], rbufs[(_CPW - 1) % 2],
                              sss[(_CPW - 1) % 2]).wait()
        plsc.subcore_barrier()
        pltpu.sync_copy(acc.at[pl.ds(r0, _RPT)], agg_hbm.at[cid, pl.ds(r0, _RPT)])

    return k(m2, dst2)


def _sc_degree(dst2, ones_c, zeros_c):
    @functools.partial(
        pl.kernel,
        out_type=jax.ShapeDtypeStruct((_NC, _NP, 8), jnp.float32),
        mesh=_MESH,
        scratch_types=[
            pltpu.VMEM((_CPW, _CH), jnp.int32),
            pltpu.VMEM((_CH, 8), jnp.float32),
            pltpu.VMEM_SHARED((_NP, 8), jnp.float32),
            pltpu.SemaphoreType.DMA,
        ],
    )
    def k(dst_hbm, ones_hbm, zeros_hbm, deg_hbm, idxb, ones, dacc, sem):
        cid = lax.axis_index("c")
        sid = lax.axis_index("s")
        wid = sid * _NC + cid
        c0 = wid * _CPW
        r0 = sid * _RPT
        pltpu.sync_copy(dst_hbm.at[pl.ds(c0, _CPW)], idxb)
        pltpu.sync_copy(ones_hbm, ones)
        pltpu.sync_copy(zeros_hbm, dacc.at[pl.ds(r0, _RPT)])
        plsc.subcore_barrier()

        def body(j, _):
            pltpu.sync_copy(ones, dacc.at[idxb.at[j]], add=True)
            return 0

        lax.fori_loop(0, _CPW, body, 0)
        plsc.subcore_barrier()
        pltpu.sync_copy(dacc.at[pl.ds(r0, _RPT)], deg_hbm.at[cid, pl.ds(r0, _RPT)])

    return k(dst2, ones_c, zeros_c)


_M1 = (np.arange(128)[:, None, None]
       == 3 * np.arange(38)[None, :, None] + np.arange(16)[None, None, :]
       ).astype(np.float32)
_M2 = (np.arange(38)[:, None, None]
       == np.arange(25)[None, :, None] + np.arange(14)[None, None, :]
       ).astype(np.float32)


def kernel(u, pos, variables, edge_index, dt, enc_W1, enc_b1, enc_W2, enc_b2,
           Wm1, bm1, Wm2, bm2, Wu1, bu1, Wu2, bu2, dec_W1, dec_b1, dec_W2, dec_b2):
    padn = _NP - _N
    u_p = jnp.pad(u, ((0, padn), (0, 0)))
    pos_p = jnp.pad(pos, ((0, padn), (0, 0)))
    var_p = jnp.pad(variables, ((0, padn), (0, 0)))

    src = edge_index[0].astype(jnp.int32)
    dst = edge_index[1].astype(jnp.int32)
    pade = _EP - _E
    fillv = jnp.full((pade,), _DUMP, jnp.int32)
    src2 = jnp.concatenate([src, fillv]).reshape(_EP // _CH, _CH)
    dst2 = jnp.concatenate([dst, fillv]).reshape(_EP // _CH, _CH)
    idx3 = jnp.stack([dst2, src2])

    w1u, w1p, w1v = enc_W1[:_TW], enc_W1[_TW:_TW + 1], enc_W1[_TW + 1:]
    b1 = enc_b1.reshape(1, _D)
    b2 = enc_b2.reshape(1, _D)

    x = _encoder(u_p, pos_p, var_p, w1u, w1p, w1v, b1, enc_W2, b2)

    w1m = jnp.einsum('hjk,ok->hoj', jnp.asarray(_M1), dec_W1[:, 0, :]).reshape(_D, 304)
    b1f = jnp.repeat(dec_b1, 38).reshape(1, 304)
    w2m = jnp.einsum('pjk,ok->opj', jnp.asarray(_M2), dec_W2[0]).reshape(304, _TW)

    stats = None
    degp = None
    y = x
    for i in range(_L):
        W = Wm1[i]
        wxi, wxj = W[:_D], W[_D:2 * _D]
        wdu = W[2 * _D:2 * _D + _TW]
        wdp = W[2 * _D + _TW:2 * _D + _TW + 1]
        wvar = W[2 * _D + _TW + 1:]
        bm = bm1[i].reshape(1, _D)
        xn, tabs = _pre(y, stats, u_p, pos_p, var_p,
                        wxi, wxj, wdu, wdp, wvar, bm)
        mpab = _sc_gather(idx3, tabs)
        m2 = _edge_mlp(mpab, Wm2[i], bm2[i].reshape(1, _D))
        aggp = _sc_scatter(m2, dst2)
        if i == 0:
            degp = _sc_degree(dst2, jnp.ones((_CH, 8), jnp.float32),
                              jnp.zeros((_RPT, 8), jnp.float32))
        Wu = Wu1[i]
        y, stats = _update(xn, aggp, degp, var_p,
                           Wu[:_D], Wu[_D:2 * _D], Wu[2 * _D:],
                           bu1[i].reshape(1, _D), Wu2[i], bu2[i].reshape(1, _D))

    out = _decoder(y, stats, u_p, dt.reshape(1, 1), w1m, b1f, w2m,
                   dec_b2.reshape(1, 1))
    return out[:_N][..., None]


# speedup vs baseline: 6.8743x; 1.0162x over previous
import functools

import jax
import jax.numpy as jnp
import numpy as np
from jax import lax
from jax.experimental import pallas as pl
from jax.experimental.pallas import tpu as pltpu
from jax.experimental.pallas import tpu_sc as plsc

_N = 10000
_E = 160000
_TW = 25
_NV = 2
_D = 128
_L = 4

_NP = 10240
_EP = 163840
_NB = 1280
_EB = 2048
_CH = 128
_NC = 2
_NS = 16
_NWORK = _NC * _NS
_CPW = _EP // (_NWORK * _CH)
_RPT = _NP // _NS
_DUMP = _N

_HIGH = lax.Precision.HIGHEST


def _silu(x):
    return x * (1.0 / (1.0 + jnp.exp(-x)))


def _dot(a, b, precision=_HIGH):
    return lax.dot_general(a, b, (((1,), (0,)), ((), ())),
                           precision=precision,
                           preferred_element_type=jnp.float32)


def _enc_body(u_ref, pos_ref, var_ref, w1u, w1p, w1v, b1, w2, b2, x_out):
    h = _dot(u_ref[...], w1u[...]) + _dot(pos_ref[...], w1p[...]) \
        + _dot(var_ref[...], w1v[...]) + b1[...]
    h = _silu(h)
    x_out[...] = _silu(_dot(h, w2[...]) + b2[...])


def _encoder(u, pos, var, w1u, w1p, w1v, b1, w2, b2):
    g = _NP // _NB
    row = lambda c: pl.BlockSpec((_NB, c), lambda i: (i, 0))
    full = lambda r, c: pl.BlockSpec((r, c), lambda i: (0, 0))
    return pl.pallas_call(
        _enc_body,
        grid=(g,),
        in_specs=[row(_TW), row(1), row(_NV), full(_TW, _D), full(1, _D),
                  full(_NV, _D), full(1, _D), full(_D, _D), full(1, _D)],
        out_specs=row(_D),
        out_shape=jax.ShapeDtypeStruct((_NP, _D), jnp.float32),
    )(u, pos, var, w1u, w1p, w1v, b1, w2, b2)


def _pack2(a):
    ah = a[:, :64].astype(jnp.bfloat16)
    al = a[:, 64:].astype(jnp.bfloat16)
    hu = lax.bitcast_convert_type(ah, jnp.uint16).astype(jnp.uint32)
    lu = lax.bitcast_convert_type(al, jnp.uint16).astype(jnp.uint32)
    return lax.bitcast_convert_type((hu << 16) | lu, jnp.float32)


def _unpack2(w):
    u = lax.bitcast_convert_type(w, jnp.uint32)
    hi = lax.bitcast_convert_type((u >> 16).astype(jnp.uint16), jnp.bfloat16)
    lo = lax.bitcast_convert_type(u.astype(jnp.uint16), jnp.bfloat16)
    return jnp.concatenate([hi, lo], axis=1).astype(jnp.float32)


def _pre_body_norm(y_ref, st_ref, u_ref, pos_ref, var_ref,
                   wxi, wxj, wdu, wdp, wvar, bm, xn_out, tab_out):
    mean = st_ref[0:1, :] * (1.0 / _N)
    ex2 = st_ref[1:2, :] * (1.0 / _N)
    rstd = lax.rsqrt(ex2 - mean * mean + 1e-5)
    xn = (y_ref[...] - mean) * rstd
    t = _dot(u_ref[...], wdu[...]) + _dot(pos_ref[...], wdp[...])
    xn_out[...] = xn
    tab_out[0] = _pack2(_dot(xn, wxi[...]) + t + _dot(var_ref[...], wvar[...])
                        + bm[...])
    tab_out[1] = _pack2(_dot(xn, wxj[...]) - t)


def _pre_body_raw(y_ref, u_ref, pos_ref, var_ref,
                  wxi, wxj, wdu, wdp, wvar, bm, xn_out, tab_out):
    xn = y_ref[...]
    t = _dot(u_ref[...], wdu[...]) + _dot(pos_ref[...], wdp[...])
    xn_out[...] = xn
    tab_out[0] = _pack2(_dot(xn, wxi[...]) + t + _dot(var_ref[...], wvar[...])
                        + bm[...])
    tab_out[1] = _pack2(_dot(xn, wxj[...]) - t)


def _pre(y, stats, u, pos, var, wxi, wxj, wdu, wdp, wvar, bm):
    g = _NP // _NB
    row = lambda c: pl.BlockSpec((_NB, c), lambda i: (i, 0))
    full = lambda r, c: pl.BlockSpec((r, c), lambda i: (0, 0))
    outs = (jax.ShapeDtypeStruct((_NP, _D), jnp.float32),
            jax.ShapeDtypeStruct((_NC, _NP, _D // 2), jnp.float32))
    out_specs = (row(_D),
                 pl.BlockSpec((_NC, _NB, _D // 2), lambda i: (0, i, 0)))
    common = [row(_TW), row(1), row(_NV), full(_D, _D), full(_D, _D),
              full(_TW, _D), full(1, _D), full(_NV, _D), full(1, _D)]
    if stats is None:
        return pl.pallas_call(
            _pre_body_raw, grid=(g,),
            in_specs=[row(_D)] + common,
            out_specs=out_specs, out_shape=outs,
        )(y, u, pos, var, wxi, wxj, wdu, wdp, wvar, bm)
    return pl.pallas_call(
        _pre_body_norm, grid=(g,),
        in_specs=[row(_D), full(8, _D)] + common,
        out_specs=out_specs, out_shape=outs,
    )(y, stats, u, pos, var, wxi, wxj, wdu, wdp, wvar, bm)


def _edge_body(ma_ref, mb_ref, w, b, out_ref):
    h = _silu(_unpack2(ma_ref[0]) + _unpack2(mb_ref[0]))
    out_ref[...] = _silu(_dot(h, w[...], precision=lax.Precision.DEFAULT) + b[...])


def _edge_mlp(mpab, w, b):
    g = _EP // _EB
    return pl.pallas_call(
        _edge_body, grid=(g,),
        in_specs=[pl.BlockSpec((1, _EB, _D // 2), lambda i: (0, i, 0)),
                  pl.BlockSpec((1, _EB, _D // 2), lambda i: (1, i, 0)),
                  pl.BlockSpec((_D, _D), lambda i: (0, 0)),
                  pl.BlockSpec((1, _D), lambda i: (0, 0))],
        out_specs=pl.BlockSpec((_EB, _D), lambda i: (i, 0)),
        out_shape=jax.ShapeDtypeStruct((_EP, _D), jnp.float32),
    )(mpab, mpab, w, b)


def _upd_body(xn_ref, a0_ref, a1_ref, d0_ref, d1_ref, var_ref,
              wux, wua, wuv, bu1, wu2, bu2, y_out, st_out):
    i = pl.program_id(0)
    deg = jnp.clip(d0_ref[0, :, 0:1] + d1_ref[0, :, 0:1], 1.0, None)
    agg = (a0_ref[0] + a1_ref[0]) / deg
    xn = xn_ref[...]
    h = _silu(_dot(xn, wux[...]) + _dot(agg, wua[...])
              + _dot(var_ref[...], wuv[...]) + bu1[...])
    y = xn + _silu(_dot(h, wu2[...]) + bu2[...])
    y_out[...] = y
    rows = i * _NB + lax.broadcasted_iota(jnp.int32, (_NB, 1), 0)
    ym = jnp.where(rows < _N, y, 0.0)
    s = jnp.sum(ym, axis=0, keepdims=True)
    ss = jnp.sum(ym * ym, axis=0, keepdims=True)
    blk = jnp.concatenate([s, ss, jnp.zeros((6, _D), jnp.float32)], axis=0)

    @pl.when(i == 0)
    def _():
        st_out[...] = blk

    @pl.when(i != 0)
    def _():
        st_out[...] = st_out[...] + blk


def _update(xn, aggp, degp, var, wux, wua, wuv, bu1, wu2, bu2):
    g = _NP // _NB
    row = lambda c: pl.BlockSpec((_NB, c), lambda i: (i, 0))
    full = lambda r, c: pl.BlockSpec((r, c), lambda i: (0, 0))
    part = lambda c, s: pl.BlockSpec((1, _NB, c), lambda i, _s=s: (_s, i, 0))
    return pl.pallas_call(
        _upd_body, grid=(g,),
        in_specs=[row(_D), part(_D, 0), part(_D, 1), part(8, 0), part(8, 1),
                  row(_NV),
                  full(_D, _D), full(_D, _D), full(_NV, _D), full(1, _D),
                  full(_D, _D), full(1, _D)],
        out_specs=(row(_D), full(8, _D)),
        out_shape=(jax.ShapeDtypeStruct((_NP, _D), jnp.float32),
                   jax.ShapeDtypeStruct((8, _D), jnp.float32)),
    )(xn, aggp, aggp, degp, degp, var, wux, wua, wuv, bu1, wu2, bu2)


def _dec_body(y_ref, st_ref, u_ref, dt_ref, w1m, b1f, w2m, b2, out_ref):
    mean = st_ref[0:1, :] * (1.0 / _N)
    ex2 = st_ref[1:2, :] * (1.0 / _N)
    rstd = lax.rsqrt(ex2 - mean * mean + 1e-5)
    xn = (y_ref[...] - mean) * rstd
    h1 = _silu(_dot(xn, w1m[...]) + b1f[...])
    diff = _dot(h1, w2m[...]) + b2[0, 0]
    tgrid = lax.broadcasted_iota(jnp.int32, (1, _TW), 1).astype(jnp.float32) + 1.0
    dtv = tgrid * dt_ref[0, 0]
    out_ref[...] = u_ref[:, _TW - 1:_TW] + dtv * diff


def _decoder(y, stats, u, dt, w1m, b1f, w2m, b2):
    g = _NP // _NB
    row = lambda c: pl.BlockSpec((_NB, c), lambda i: (i, 0))
    full = lambda r, c: pl.BlockSpec((r, c), lambda i: (0, 0))
    return pl.pallas_call(
        _dec_body, grid=(g,),
        in_specs=[row(_D), full(8, _D), row(_TW), full(1, 1),
                  full(_D, 304), full(1, 304), full(304, _TW), full(1, 1)],
        out_specs=row(_TW),
        out_shape=jax.ShapeDtypeStruct((_NP, _TW), jnp.float32),
    )(y, stats, u, dt, w1m, b1f, w2m, b2)


_MESH = plsc.VectorSubcoreMesh(core_axis_name="c", subcore_axis_name="s")


_HD = _D // 2
_CPT = _EP // _CH // _NS


def _sc_gather(idx3, tabs):
    @functools.partial(
        pl.kernel,
        out_type=jax.ShapeDtypeStruct((_NC, _EP, _HD), jnp.float32),
        mesh=_MESH,
        scratch_types=[
            pltpu.VMEM((_CPT, _CH), jnp.int32),
            pltpu.VMEM((_CH, _HD), jnp.float32),
            pltpu.VMEM((_CH, _HD), jnp.float32),
            pltpu.VMEM_SHARED((_NP, _HD), jnp.float32),
            pltpu.SemaphoreType.DMA,
            pltpu.SemaphoreType.DMA,
        ],
    )
    def k(idx_hbm, tab_hbm, out_hbm, idxb, r0b, r1b, table, sw0, sw1):
        rbufs, sws = (r0b, r1b), (sw0, sw1)
        cid = lax.axis_index("c")
        sid = lax.axis_index("s")
        c0 = sid * _CPT
        pltpu.sync_copy(idx_hbm.at[cid, pl.ds(c0, _CPT)], idxb)
        pltpu.sync_copy(tab_hbm.at[cid, pl.ds(sid * _RPT, _RPT)],
                        table.at[pl.ds(sid * _RPT, _RPT)])
        plsc.subcore_barrier()

        def outer(g, _):
            for b in range(2):
                j = g * 2 + b
                rows = rbufs[b]

                @pl.when(j >= 2)
                def _():
                    pltpu.make_async_copy(out_hbm.at[cid, pl.ds(0, _CH)],
                                          rows, sws[b]).wait()

                pltpu.sync_copy(table.at[idxb.at[j]], rows)
                pltpu.async_copy(
                    rows, out_hbm.at[cid, pl.ds((c0 + j) * _CH, _CH)], sws[b])
            return 0

        lax.fori_loop(0, _CPT // 2, outer, 0)
        for b in range(2):
            pltpu.make_async_copy(out_hbm.at[cid, pl.ds(0, _CH)], rbufs[b],
                                  sws[b]).wait()

    return k(idx3, tabs)


def _zero_vmem(ref, rows, cols):
    z = jnp.zeros((16,), jnp.float32)

    def zr(r, _):
        for q in range(cols // 16):
            ref[r, pl.ds(q * 16, 16)] = z
        return 0

    lax.fori_loop(0, rows, zr, 0)


def _sc_scatter(m2, dst2):
    @functools.partial(
        pl.kernel,
        out_type=jax.ShapeDtypeStruct((_NC, _NP, _D), jnp.float32),
        mesh=_MESH,
        scratch_types=[
            pltpu.VMEM((_CPW, _CH), jnp.int32),
            pltpu.VMEM((_CH, _D), jnp.float32),
            pltpu.VMEM((_CH, _D), jnp.float32),
            pltpu.VMEM_SHARED((_NP, _D), jnp.float32),
            pltpu.SemaphoreType.DMA,
            pltpu.SemaphoreType.DMA,
            pltpu.SemaphoreType.DMA,
            pltpu.SemaphoreType.DMA,
        ],
    )
    def k(m2_hbm, dst_hbm, agg_hbm, idxb, r0buf, r1buf, acc, sl0, sl1, ss0, ss1):
        rbufs, sls, sss = (r0buf, r1buf), (sl0, sl1), (ss0, ss1)
        cid = lax.axis_index("c")
        sid = lax.axis_index("s")
        wid = sid * _NC + cid
        c0 = wid * _CPW
        r0 = sid * _RPT
        pltpu.sync_copy(dst_hbm.at[pl.ds(c0, _CPW)], idxb)
        _zero_vmem(r0buf, _CH, _D)
        for t in range(_RPT // _CH):
            pltpu.sync_copy(r0buf, acc.at[pl.ds(r0 + t * _CH, _CH)])
        plsc.subcore_barrier()
        pltpu.async_copy(m2_hbm.at[pl.ds(c0 * _CH, _CH)], rbufs[0], sls[0])

        def outer(g, _):
            for b in range(2):
                j = g * 2 + b
                ob = 1 - b
                rows = rbufs[b]
                pltpu.make_async_copy(m2_hbm.at[pl.ds(0, _CH)], rows,
                                      sls[b]).wait()
                pltpu.async_copy(rows, acc.at[idxb.at[j]], sss[b], add=True)

                @pl.when(j >= 1)
                def _():
                    pltpu.make_async_copy(acc.at[pl.ds(0, _CH)], rbufs[ob],
                                          sss[ob]).wait()

                @pl.when(j + 1 < _CPW)
                def _():
                    pltpu.async_copy(m2_hbm.at[pl.ds((c0 + j + 1) * _CH, _CH)],
                                     rbufs[ob], sls[ob])

            return 0

        lax.fori_loop(0, _CPW // 2, outer, 0)
        pltpu.make_async_copy(acc.at[pl.ds(0, _CH)], rbufs[(_CPW - 1) % 2],
                              sss[(_CPW - 1) % 2]).wait()
        plsc.subcore_barrier()
        pltpu.sync_copy(acc.at[pl.ds(r0, _RPT)], agg_hbm.at[cid, pl.ds(r0, _RPT)])

    return k(m2, dst2)


def _sc_degree(dst2, ones_c, zeros_c):
    @functools.partial(
        pl.kernel,
        out_type=jax.ShapeDtypeStruct((_NC, _NP, 8), jnp.float32),
        mesh=_MESH,
        scratch_types=[
            pltpu.VMEM((_CPW, _CH), jnp.int32),
            pltpu.VMEM((_CH, 8), jnp.float32),
            pltpu.VMEM_SHARED((_NP, 8), jnp.float32),
            pltpu.SemaphoreType.DMA,
        ],
    )
    def k(dst_hbm, ones_hbm, zeros_hbm, deg_hbm, idxb, ones, dacc, sem):
        cid = lax.axis_index("c")
        sid = lax.axis_index("s")
        wid = sid * _NC + cid
        c0 = wid * _CPW
        r0 = sid * _RPT
        pltpu.sync_copy(dst_hbm.at[pl.ds(c0, _CPW)], idxb)
        pltpu.sync_copy(ones_hbm, ones)
        pltpu.sync_copy(zeros_hbm, dacc.at[pl.ds(r0, _RPT)])
        plsc.subcore_barrier()

        def body(j, _):
            pltpu.sync_copy(ones, dacc.at[idxb.at[j]], add=True)
            return 0

        lax.fori_loop(0, _CPW, body, 0)
        plsc.subcore_barrier()
        pltpu.sync_copy(dacc.at[pl.ds(r0, _RPT)], deg_hbm.at[cid, pl.ds(r0, _RPT)])

    return k(dst2, ones_c, zeros_c)


_M1 = (np.arange(128)[:, None, None]
       == 3 * np.arange(38)[None, :, None] + np.arange(16)[None, None, :]
       ).astype(np.float32)
_M2 = (np.arange(38)[:, None, None]
       == np.arange(25)[None, :, None] + np.arange(14)[None, None, :]
       ).astype(np.float32)


def kernel(u, pos, variables, edge_index, dt, enc_W1, enc_b1, enc_W2, enc_b2,
           Wm1, bm1, Wm2, bm2, Wu1, bu1, Wu2, bu2, dec_W1, dec_b1, dec_W2, dec_b2):
    padn = _NP - _N
    u_p = jnp.pad(u, ((0, padn), (0, 0)))
    pos_p = jnp.pad(pos, ((0, padn), (0, 0)))
    var_p = jnp.pad(variables, ((0, padn), (0, 0)))

    src = edge_index[0].astype(jnp.int32)
    dst = edge_index[1].astype(jnp.int32)
    pade = _EP - _E
    fillv = jnp.full((pade,), _DUMP, jnp.int32)
    src2 = jnp.concatenate([src, fillv]).reshape(_EP // _CH, _CH)
    dst2 = jnp.concatenate([dst, fillv]).reshape(_EP // _CH, _CH)
    idx3 = jnp.stack([dst2, src2])

    w1u, w1p, w1v = enc_W1[:_TW], enc_W1[_TW:_TW + 1], enc_W1[_TW + 1:]
    b1 = enc_b1.reshape(1, _D)
    b2 = enc_b2.reshape(1, _D)

    x = _encoder(u_p, pos_p, var_p, w1u, w1p, w1v, b1, enc_W2, b2)

    w1m = jnp.einsum('hjk,ok->hoj', jnp.asarray(_M1), dec_W1[:, 0, :]).reshape(_D, 304)
    b1f = jnp.repeat(dec_b1, 38).reshape(1, 304)
    w2m = jnp.einsum('pjk,ok->opj', jnp.asarray(_M2), dec_W2[0]).reshape(304, _TW)

    stats = None
    degp = None
    y = x
    for i in range(_L):
        W = Wm1[i]
        wxi, wxj = W[:_D], W[_D:2 * _D]
        wdu = W[2 * _D:2 * _D + _TW]
        wdp = W[2 * _D + _TW:2 * _D + _TW + 1]
        wvar = W[2 * _D + _TW + 1:]
        bm = bm1[i].reshape(1, _D)
        xn, tabs = _pre(y, stats, u_p, pos_p, var_p,
                        wxi, wxj, wdu, wdp, wvar, bm)
        mpab = _sc_gather(idx3, tabs)
        m2 = _edge_mlp(mpab, Wm2[i], bm2[i].reshape(1, _D))
        aggp = _sc_scatter(m2, dst2)
        if i == 0:
            degp = _sc_degree(dst2, jnp.ones((_CH, 8), jnp.float32),
                              jnp.zeros((_RPT, 8), jnp.float32))
        Wu = Wu1[i]
        y, stats = _update(xn, aggp, degp, var_p,
                           Wu[:_D], Wu[_D:2 * _D], Wu[2 * _D:],
                           bu1[i].reshape(1, _D), Wu2[i], bu2[i].reshape(1, _D))

    out = _decoder(y, stats, u_p, dt.reshape(1, 1), w1m, b1f, w2m,
                   dec_b2.reshape(1, 1))
    return out[:_N][..., None]


# speedup vs baseline: 7.7784x; 1.1315x over previous
import functools

import jax
import jax.numpy as jnp
import numpy as np
from jax import lax
from jax.experimental import pallas as pl
from jax.experimental.pallas import tpu as pltpu
from jax.experimental.pallas import tpu_sc as plsc

_N = 10000
_E = 160000
_TW = 25
_NV = 2
_D = 128
_L = 4

_NP = 10240
_EP = 163840
_NB = 1280
_EB = 2048
_CH = 128
_NC = 2
_NS = 16
_NWORK = _NC * _NS
_CPW = _EP // (_NWORK * _CH)
_RPT = _NP // _NS
_DUMP = _N

_HIGH = lax.Precision.DEFAULT


def _silu(x):
    return x * (1.0 / (1.0 + jnp.exp(-x)))


def _dot(a, b, precision=_HIGH):
    return lax.dot_general(a, b, (((1,), (0,)), ((), ())),
                           precision=precision,
                           preferred_element_type=jnp.float32)


def _enc_body(u_ref, pos_ref, var_ref, w1u, w1p, w1v, b1, w2, b2, x_out):
    h = _dot(u_ref[...], w1u[...]) + _dot(pos_ref[...], w1p[...]) \
        + _dot(var_ref[...], w1v[...]) + b1[...]
    h = _silu(h)
    x_out[...] = _silu(_dot(h, w2[...]) + b2[...])


def _encoder(u, pos, var, w1u, w1p, w1v, b1, w2, b2):
    g = _NP // _NB
    row = lambda c: pl.BlockSpec((_NB, c), lambda i: (i, 0))
    full = lambda r, c: pl.BlockSpec((r, c), lambda i: (0, 0))
    return pl.pallas_call(
        _enc_body,
        grid=(g,),
        in_specs=[row(_TW), row(1), row(_NV), full(_TW, _D), full(1, _D),
                  full(_NV, _D), full(1, _D), full(_D, _D), full(1, _D)],
        out_specs=row(_D),
        out_shape=jax.ShapeDtypeStruct((_NP, _D), jnp.float32),
    )(u, pos, var, w1u, w1p, w1v, b1, w2, b2)


def _pack2(a):
    ah = a[:, :64].astype(jnp.bfloat16)
    al = a[:, 64:].astype(jnp.bfloat16)
    hu = lax.bitcast_convert_type(ah, jnp.uint16).astype(jnp.uint32)
    lu = lax.bitcast_convert_type(al, jnp.uint16).astype(jnp.uint32)
    return lax.bitcast_convert_type((hu << 16) | lu, jnp.float32)


def _unpack2(w):
    u = lax.bitcast_convert_type(w, jnp.uint32)
    hi = lax.bitcast_convert_type((u >> 16).astype(jnp.uint16), jnp.bfloat16)
    lo = lax.bitcast_convert_type(u.astype(jnp.uint16), jnp.bfloat16)
    return jnp.concatenate([hi, lo], axis=1).astype(jnp.float32)


def _pre_body_norm(y_ref, st_ref, u_ref, pos_ref, var_ref,
                   wxi, wxj, wdu, wdp, wvar, bm, xn_out, tab_out):
    mean = st_ref[0:1, :] * (1.0 / _N)
    ex2 = st_ref[1:2, :] * (1.0 / _N)
    rstd = lax.rsqrt(ex2 - mean * mean + 1e-5)
    xn = (y_ref[...] - mean) * rstd
    t = _dot(u_ref[...], wdu[...]) + _dot(pos_ref[...], wdp[...])
    xn_out[...] = xn
    tab_out[0] = _pack2(_dot(xn, wxi[...]) + t + _dot(var_ref[...], wvar[...])
                        + bm[...])
    tab_out[1] = _pack2(_dot(xn, wxj[...]) - t)


def _pre_body_raw(y_ref, u_ref, pos_ref, var_ref,
                  wxi, wxj, wdu, wdp, wvar, bm, xn_out, tab_out):
    xn = y_ref[...]
    t = _dot(u_ref[...], wdu[...]) + _dot(pos_ref[...], wdp[...])
    xn_out[...] = xn
    tab_out[0] = _pack2(_dot(xn, wxi[...]) + t + _dot(var_ref[...], wvar[...])
                        + bm[...])
    tab_out[1] = _pack2(_dot(xn, wxj[...]) - t)


def _pre(y, stats, u, pos, var, wxi, wxj, wdu, wdp, wvar, bm):
    g = _NP // _NB
    row = lambda c: pl.BlockSpec((_NB, c), lambda i: (i, 0))
    full = lambda r, c: pl.BlockSpec((r, c), lambda i: (0, 0))
    outs = (jax.ShapeDtypeStruct((_NP, _D), jnp.float32),
            jax.ShapeDtypeStruct((_NC, _NP, _D // 2), jnp.float32))
    out_specs = (row(_D),
                 pl.BlockSpec((_NC, _NB, _D // 2), lambda i: (0, i, 0)))
    common = [row(_TW), row(1), row(_NV), full(_D, _D), full(_D, _D),
              full(_TW, _D), full(1, _D), full(_NV, _D), full(1, _D)]
    if stats is None:
        return pl.pallas_call(
            _pre_body_raw, grid=(g,),
            in_specs=[row(_D)] + common,
            out_specs=out_specs, out_shape=outs,
        )(y, u, pos, var, wxi, wxj, wdu, wdp, wvar, bm)
    return pl.pallas_call(
        _pre_body_norm, grid=(g,),
        in_specs=[row(_D), full(8, _D)] + common,
        out_specs=out_specs, out_shape=outs,
    )(y, stats, u, pos, var, wxi, wxj, wdu, wdp, wvar, bm)


def _edge_body(ma_ref, mb_ref, w, b, out_ref):
    h = _silu(_unpack2(ma_ref[0]) + _unpack2(mb_ref[0]))
    out_ref[...] = _silu(_dot(h, w[...], precision=lax.Precision.DEFAULT) + b[...])


def _edge_mlp(mpab, w, b):
    g = _EP // _EB
    return pl.pallas_call(
        _edge_body, grid=(g,),
        in_specs=[pl.BlockSpec((1, _EB, _D // 2), lambda i: (0, i, 0)),
                  pl.BlockSpec((1, _EB, _D // 2), lambda i: (1, i, 0)),
                  pl.BlockSpec((_D, _D), lambda i: (0, 0)),
                  pl.BlockSpec((1, _D), lambda i: (0, 0))],
        out_specs=pl.BlockSpec((_EB, _D), lambda i: (i, 0)),
        out_shape=jax.ShapeDtypeStruct((_EP, _D), jnp.float32),
    )(mpab, mpab, w, b)


def _upd_body(xn_ref, a0_ref, a1_ref, d0_ref, d1_ref, var_ref,
              wux, wua, wuv, bu1, wu2, bu2, y_out, st_out):
    i = pl.program_id(0)
    deg = jnp.clip(d0_ref[0, :, 0:1] + d1_ref[0, :, 0:1], 1.0, None)
    agg = (a0_ref[0] + a1_ref[0]) / deg
    xn = xn_ref[...]
    h = _silu(_dot(xn, wux[...]) + _dot(agg, wua[...])
              + _dot(var_ref[...], wuv[...]) + bu1[...])
    y = xn + _silu(_dot(h, wu2[...]) + bu2[...])
    y_out[...] = y
    rows = i * _NB + lax.broadcasted_iota(jnp.int32, (_NB, 1), 0)
    ym = jnp.where(rows < _N, y, 0.0)
    s = jnp.sum(ym, axis=0, keepdims=True)
    ss = jnp.sum(ym * ym, axis=0, keepdims=True)
    blk = jnp.concatenate([s, ss, jnp.zeros((6, _D), jnp.float32)], axis=0)

    @pl.when(i == 0)
    def _():
        st_out[...] = blk

    @pl.when(i != 0)
    def _():
        st_out[...] = st_out[...] + blk


def _update(xn, aggp, degp, var, wux, wua, wuv, bu1, wu2, bu2):
    g = _NP // _NB
    row = lambda c: pl.BlockSpec((_NB, c), lambda i: (i, 0))
    full = lambda r, c: pl.BlockSpec((r, c), lambda i: (0, 0))
    part = lambda c, s: pl.BlockSpec((1, _NB, c), lambda i, _s=s: (_s, i, 0))
    return pl.pallas_call(
        _upd_body, grid=(g,),
        in_specs=[row(_D), part(_D, 0), part(_D, 1), part(8, 0), part(8, 1),
                  row(_NV),
                  full(_D, _D), full(_D, _D), full(_NV, _D), full(1, _D),
                  full(_D, _D), full(1, _D)],
        out_specs=(row(_D), full(8, _D)),
        out_shape=(jax.ShapeDtypeStruct((_NP, _D), jnp.float32),
                   jax.ShapeDtypeStruct((8, _D), jnp.float32)),
    )(xn, aggp, aggp, degp, degp, var, wux, wua, wuv, bu1, wu2, bu2)


def _dec_body(y_ref, st_ref, u_ref, dt_ref, w1m, b1f, w2m, b2, out_ref):
    mean = st_ref[0:1, :] * (1.0 / _N)
    ex2 = st_ref[1:2, :] * (1.0 / _N)
    rstd = lax.rsqrt(ex2 - mean * mean + 1e-5)
    xn = (y_ref[...] - mean) * rstd
    h1 = _silu(_dot(xn, w1m[...]) + b1f[...])
    diff = _dot(h1, w2m[...]) + b2[0, 0]
    tgrid = lax.broadcasted_iota(jnp.int32, (1, _TW), 1).astype(jnp.float32) + 1.0
    dtv = tgrid * dt_ref[0, 0]
    out_ref[...] = u_ref[:, _TW - 1:_TW] + dtv * diff


def _decoder(y, stats, u, dt, w1m, b1f, w2m, b2):
    g = _NP // _NB
    row = lambda c: pl.BlockSpec((_NB, c), lambda i: (i, 0))
    full = lambda r, c: pl.BlockSpec((r, c), lambda i: (0, 0))
    return pl.pallas_call(
        _dec_body, grid=(g,),
        in_specs=[row(_D), full(8, _D), row(_TW), full(1, 1),
                  full(_D, 304), full(1, 304), full(304, _TW), full(1, 1)],
        out_specs=row(_TW),
        out_shape=jax.ShapeDtypeStruct((_NP, _TW), jnp.float32),
    )(y, stats, u, dt, w1m, b1f, w2m, b2)


_MESH = plsc.VectorSubcoreMesh(core_axis_name="c", subcore_axis_name="s")


_HD = _D // 2
_CPT = _EP // _CH // _NS


def _sc_gather(idx3, tabs):
    @functools.partial(
        pl.kernel,
        out_type=jax.ShapeDtypeStruct((_NC, _EP, _HD), jnp.float32),
        mesh=_MESH,
        scratch_types=[
            pltpu.VMEM((_CPT, _CH), jnp.int32),
            pltpu.VMEM((_CH, _HD), jnp.float32),
            pltpu.VMEM((_CH, _HD), jnp.float32),
            pltpu.VMEM_SHARED((_NP, _HD), jnp.float32),
            pltpu.SemaphoreType.DMA,
            pltpu.SemaphoreType.DMA,
        ],
    )
    def k(idx_hbm, tab_hbm, out_hbm, idxb, r0b, r1b, table, sw0, sw1):
        rbufs, sws = (r0b, r1b), (sw0, sw1)
        cid = lax.axis_index("c")
        sid = lax.axis_index("s")
        c0 = sid * _CPT
        pltpu.sync_copy(idx_hbm.at[cid, pl.ds(c0, _CPT)], idxb)
        pltpu.sync_copy(tab_hbm.at[cid, pl.ds(sid * _RPT, _RPT)],
                        table.at[pl.ds(sid * _RPT, _RPT)])
        plsc.subcore_barrier()

        def outer(g, _):
            for b in range(2):
                j = g * 2 + b
                rows = rbufs[b]

                @pl.when(j >= 2)
                def _():
                    pltpu.make_async_copy(out_hbm.at[cid, pl.ds(0, _CH)],
                                          rows, sws[b]).wait()

                pltpu.sync_copy(table.at[idxb.at[j]], rows)
                pltpu.async_copy(
                    rows, out_hbm.at[cid, pl.ds((c0 + j) * _CH, _CH)], sws[b])
            return 0

        lax.fori_loop(0, _CPT // 2, outer, 0)
        for b in range(2):
            pltpu.make_async_copy(out_hbm.at[cid, pl.ds(0, _CH)], rbufs[b],
                                  sws[b]).wait()

    return k(idx3, tabs)


def _zero_vmem(ref, rows, cols):
    z = jnp.zeros((16,), jnp.float32)

    def zr(r, _):
        for q in range(cols // 16):
            ref[r, pl.ds(q * 16, 16)] = z
        return 0

    lax.fori_loop(0, rows, zr, 0)


def _sc_scatter(m2, dst2):
    @functools.partial(
        pl.kernel,
        out_type=jax.ShapeDtypeStruct((_NC, _NP, _D), jnp.float32),
        mesh=_MESH,
        scratch_types=[
            pltpu.VMEM((_CPW, _CH), jnp.int32),
            pltpu.VMEM((_CH, _D), jnp.float32),
            pltpu.VMEM((_CH, _D), jnp.float32),
            pltpu.VMEM_SHARED((_NP, _D), jnp.float32),
            pltpu.SemaphoreType.DMA,
            pltpu.SemaphoreType.DMA,
            pltpu.SemaphoreType.DMA,
            pltpu.SemaphoreType.DMA,
        ],
    )
    def k(m2_hbm, dst_hbm, agg_hbm, idxb, r0buf, r1buf, acc, sl0, sl1, ss0, ss1):
        rbufs, sls, sss = (r0buf, r1buf), (sl0, sl1), (ss0, ss1)
        cid = lax.axis_index("c")
        sid = lax.axis_index("s")
        wid = sid * _NC + cid
        c0 = wid * _CPW
        r0 = sid * _RPT
        pltpu.sync_copy(dst_hbm.at[pl.ds(c0, _CPW)], idxb)
        _zero_vmem(r0buf, _CH, _D)
        for t in range(_RPT // _CH):
            pltpu.sync_copy(r0buf, acc.at[pl.ds(r0 + t * _CH, _CH)])
        plsc.subcore_barrier()
        pltpu.async_copy(m2_hbm.at[pl.ds(c0 * _CH, _CH)], rbufs[0], sls[0])

        def outer(g, _):
            for b in range(2):
                j = g * 2 + b
                ob = 1 - b
                rows = rbufs[b]
                pltpu.make_async_copy(m2_hbm.at[pl.ds(0, _CH)], rows,
                                      sls[b]).wait()
                pltpu.async_copy(rows, acc.at[idxb.at[j]], sss[b], add=True)

                @pl.when(j >= 1)
                def _():
                    pltpu.make_async_copy(acc.at[pl.ds(0, _CH)], rbufs[ob],
                                          sss[ob]).wait()

                @pl.when(j + 1 < _CPW)
                def _():
                    pltpu.async_copy(m2_hbm.at[pl.ds((c0 + j + 1) * _CH, _CH)],
                                     rbufs[ob], sls[ob])

            return 0

        lax.fori_loop(0, _CPW // 2, outer, 0)
        pltpu.make_async_copy(acc.at[pl.ds(0, _CH)], rbufs[(_CPW - 1) % 2],
                              sss[(_CPW - 1) % 2]).wait()
        plsc.subcore_barrier()
        pltpu.sync_copy(acc.at[pl.ds(r0, _RPT)], agg_hbm.at[cid, pl.ds(r0, _RPT)])

    return k(m2, dst2)


def _sc_degree(dst2, ones_c, zeros_c):
    @functools.partial(
        pl.kernel,
        out_type=jax.ShapeDtypeStruct((_NC, _NP, 8), jnp.float32),
        mesh=_MESH,
        scratch_types=[
            pltpu.VMEM((_CPW, _CH), jnp.int32),
            pltpu.VMEM((_CH, 8), jnp.float32),
            pltpu.VMEM_SHARED((_NP, 8), jnp.float32),
            pltpu.SemaphoreType.DMA,
        ],
    )
    def k(dst_hbm, ones_hbm, zeros_hbm, deg_hbm, idxb, ones, dacc, sem):
        cid = lax.axis_index("c")
        sid = lax.axis_index("s")
        wid = sid * _NC + cid
        c0 = wid * _CPW
        r0 = sid * _RPT
        pltpu.sync_copy(dst_hbm.at[pl.ds(c0, _CPW)], idxb)
        pltpu.sync_copy(ones_hbm, ones)
        pltpu.sync_copy(zeros_hbm, dacc.at[pl.ds(r0, _RPT)])
        plsc.subcore_barrier()

        def body(j, _):
            pltpu.sync_copy(ones, dacc.at[idxb.at[j]], add=True)
            return 0

        lax.fori_loop(0, _CPW, body, 0)
        plsc.subcore_barrier()
        pltpu.sync_copy(dacc.at[pl.ds(r0, _RPT)], deg_hbm.at[cid, pl.ds(r0, _RPT)])

    return k(dst2, ones_c, zeros_c)


_M1 = (np.arange(128)[:, None, None]
       == 3 * np.arange(38)[None, :, None] + np.arange(16)[None, None, :]
       ).astype(np.float32)
_M2 = (np.arange(38)[:, None, None]
       == np.arange(25)[None, :, None] + np.arange(14)[None, None, :]
       ).astype(np.float32)


def kernel(u, pos, variables, edge_index, dt, enc_W1, enc_b1, enc_W2, enc_b2,
           Wm1, bm1, Wm2, bm2, Wu1, bu1, Wu2, bu2, dec_W1, dec_b1, dec_W2, dec_b2):
    padn = _NP - _N
    u_p = jnp.pad(u, ((0, padn), (0, 0)))
    pos_p = jnp.pad(pos, ((0, padn), (0, 0)))
    var_p = jnp.pad(variables, ((0, padn), (0, 0)))

    src = edge_index[0].astype(jnp.int32)
    dst = edge_index[1].astype(jnp.int32)
    pade = _EP - _E
    fillv = jnp.full((pade,), _DUMP, jnp.int32)
    src2 = jnp.concatenate([src, fillv]).reshape(_EP // _CH, _CH)
    dst2 = jnp.concatenate([dst, fillv]).reshape(_EP // _CH, _CH)
    idx3 = jnp.stack([dst2, src2])

    w1u, w1p, w1v = enc_W1[:_TW], enc_W1[_TW:_TW + 1], enc_W1[_TW + 1:]
    b1 = enc_b1.reshape(1, _D)
    b2 = enc_b2.reshape(1, _D)

    x = _encoder(u_p, pos_p, var_p, w1u, w1p, w1v, b1, enc_W2, b2)

    w1m = jnp.einsum('hjk,ok->hoj', jnp.asarray(_M1), dec_W1[:, 0, :]).reshape(_D, 304)
    b1f = jnp.repeat(dec_b1, 38).reshape(1, 304)
    w2m = jnp.einsum('pjk,ok->opj', jnp.asarray(_M2), dec_W2[0]).reshape(304, _TW)

    stats = None
    degp = None
    y = x
    for i in range(_L):
        W = Wm1[i]
        wxi, wxj = W[:_D], W[_D:2 * _D]
        wdu = W[2 * _D:2 * _D + _TW]
        wdp = W[2 * _D + _TW:2 * _D + _TW + 1]
        wvar = W[2 * _D + _TW + 1:]
        bm = bm1[i].reshape(1, _D)
        xn, tabs = _pre(y, stats, u_p, pos_p, var_p,
                        wxi, wxj, wdu, wdp, wvar, bm)
        mpab = _sc_gather(idx3, tabs)
        m2 = _edge_mlp(mpab, Wm2[i], bm2[i].reshape(1, _D))
        aggp = _sc_scatter(m2, dst2)
        if i == 0:
            degp = _sc_degree(dst2, jnp.ones((_CH, 8), jnp.float32),
                              jnp.zeros((_RPT, 8), jnp.float32))
        Wu = Wu1[i]
        y, stats = _update(xn, aggp, degp, var_p,
                           Wu[:_D], Wu[_D:2 * _D], Wu[2 * _D:],
                           bu1[i].reshape(1, _D), Wu2[i], bu2[i].reshape(1, _D))

    out = _decoder(y, stats, u_p, dt.reshape(1, 1), w1m, b1f, w2m,
                   dec_b2.reshape(1, 1))
    return out[:_N][..., None]
